# Initial kernel scaffold; baseline (speedup 1.0000x reference)
import jax
import jax.numpy as jnp
from jax.experimental import pallas as pl


def kernel(x, edge_index, batch, emb_charge, emb_pdgid, emb_frompv, W_cat, b_cat, W_cont, b_cont, W_all, b_all, bn_gamma, bn_beta, W1_0, b1_0, W2_0, b2_0, W1_1, b1_1, W2_1, b2_1, W_out1, b_out1, W_out2, b_out2):
    raise NotImplementedError("write your pallas kernel here")


# speedup vs baseline: 1.0009x; 1.0009x over previous
import functools

import jax
import jax.numpy as jnp
from jax import lax
from jax.experimental import pallas as pl
from jax.experimental.pallas import tpu as pltpu

N = 50000
E = 800000
H = 32
MESG = 64
PDGS = (1, 2, 11, 13, 22, 130, 211)


def _elu(v):
    return jnp.where(v > 0, v, jnp.exp(jnp.minimum(v, 0.0)) - 1.0)


def _encode_body(x_ref, wcat_ref, bcat_ref, wcont_ref, bcont_ref,
                 wall_ref, ball_ref, h_ref, s1_ref, s2_ref):
    i = pl.program_id(0)
    x = x_ref[...]
    x_cont = x[:, :8]
    pdgv = jnp.abs(x[:, 8:9])
    chv = x[:, 9:10] + 1.0
    fpv = x[:, 10:11]
    oh_ch = jnp.concatenate(
        [(chv == float(v)).astype(jnp.float32) for v in range(3)], axis=1)
    oh_pdg = jnp.concatenate(
        [(pdgv == float(v)).astype(jnp.float32) for v in PDGS], axis=1)
    oh_fp = jnp.concatenate(
        [(fpv == float(v)).astype(jnp.float32) for v in range(4)], axis=1)
    oh = jnp.concatenate([oh_ch, oh_pdg, oh_fp], axis=1)
    emb_cat = _elu(jnp.dot(oh, wcat_ref[...], preferred_element_type=jnp.float32)
                   + bcat_ref[...])
    emb_cont = _elu(jnp.dot(x_cont, wcont_ref[...], preferred_element_type=jnp.float32)
                    + bcont_ref[...])
    hin = jnp.concatenate([emb_cat, emb_cont], axis=1)
    h = _elu(jnp.dot(hin, wall_ref[...], preferred_element_type=jnp.float32)
             + ball_ref[...])
    h_ref[...] = h

    @pl.when(i == 0)
    def _():
        s1_ref[...] = jnp.zeros_like(s1_ref)
        s2_ref[...] = jnp.zeros_like(s2_ref)

    s1_ref[...] += jnp.sum(h, axis=0, keepdims=True)
    s2_ref[...] += jnp.sum(h * h, axis=0, keepdims=True)


def _encode(x, W_cat_eff, b_cat, W_cont, b_cont, W_all, b_all):
    blk = 5000
    grid = N // blk
    full = lambda s: pl.BlockSpec(s, lambda i: (0,) * len(s))
    return pl.pallas_call(
        _encode_body,
        grid=(grid,),
        in_specs=[
            pl.BlockSpec((blk, 11), lambda i: (i, 0)),
            full((14, 16)), full((1, 16)), full((8, 16)), full((1, 16)),
            full((32, 32)), full((1, 32)),
        ],
        out_specs=[
            pl.BlockSpec((blk, 32), lambda i: (i, 0)),
            full((1, 32)), full((1, 32)),
        ],
        out_shape=[
            jax.ShapeDtypeStruct((N, 32), jnp.float32),
            jax.ShapeDtypeStruct((1, 32), jnp.float32),
            jax.ShapeDtypeStruct((1, 32), jnp.float32),
        ],
    )(x, W_cat_eff, b_cat, W_cont, b_cont, W_all, b_all)


def _pq_body(h_ref, s1_ref, s2_ref, g_ref, b_ref, wp_ref, wq_ref, bp_ref,
             h_out_ref, p_ref, q_ref):
    mean = s1_ref[...] / N
    var = s2_ref[...] / N - mean * mean
    inv = g_ref[...] * lax.rsqrt(var + 1e-5)
    h = h_ref[...] * inv + (b_ref[...] - mean * inv)
    h_out_ref[...] = h
    p_ref[...] = jnp.dot(h, wp_ref[...], preferred_element_type=jnp.float32) + bp_ref[...]
    q_ref[...] = jnp.dot(h, wq_ref[...], preferred_element_type=jnp.float32)


def _pq_first(h0, s1, s2, gamma, beta, WP, WQ, b1):
    blk = 5000
    full = lambda s: pl.BlockSpec(s, lambda i: (0,) * len(s))
    return pl.pallas_call(
        _pq_body,
        grid=(N // blk,),
        in_specs=[
            pl.BlockSpec((blk, 32), lambda i: (i, 0)),
            full((1, 32)), full((1, 32)), full((1, 32)), full((1, 32)),
            full((32, 64)), full((32, 64)), full((1, 64)),
        ],
        out_specs=[
            pl.BlockSpec((blk, 32), lambda i: (i, 0)),
            pl.BlockSpec((blk, 64), lambda i: (i, 0)),
            pl.BlockSpec((blk, 64), lambda i: (i, 0)),
        ],
        out_shape=[
            jax.ShapeDtypeStruct((N, 32), jnp.float32),
            jax.ShapeDtypeStruct((N, 64), jnp.float32),
            jax.ShapeDtypeStruct((N, 64), jnp.float32),
        ],
    )(h0, s1, s2, gamma, beta, WP, WQ, b1)


def _pq_next_body(h_ref, agg_ref, wp_ref, wq_ref, bp_ref,
                  h_out_ref, p_ref, q_ref):
    h = h_ref[...] + agg_ref[...]
    h_out_ref[...] = h
    p_ref[...] = jnp.dot(h, wp_ref[...], preferred_element_type=jnp.float32) + bp_ref[...]
    q_ref[...] = jnp.dot(h, wq_ref[...], preferred_element_type=jnp.float32)


def _pq_next(h, agg, WP, WQ, b1):
    blk = 5000
    full = lambda s: pl.BlockSpec(s, lambda i: (0,) * len(s))
    return pl.pallas_call(
        _pq_next_body,
        grid=(N // blk,),
        in_specs=[
            pl.BlockSpec((blk, 32), lambda i: (i, 0)),
            pl.BlockSpec((blk, 32), lambda i: (i, 0)),
            full((32, 64)), full((32, 64)), full((1, 64)),
        ],
        out_specs=[
            pl.BlockSpec((blk, 32), lambda i: (i, 0)),
            pl.BlockSpec((blk, 64), lambda i: (i, 0)),
            pl.BlockSpec((blk, 64), lambda i: (i, 0)),
        ],
        out_shape=[
            jax.ShapeDtypeStruct((N, 32), jnp.float32),
            jax.ShapeDtypeStruct((N, 64), jnp.float32),
            jax.ShapeDtypeStruct((N, 64), jnp.float32),
        ],
    )(h, agg, WP, WQ, b1)


def _mm_body(g_ref, w2_ref, b2_ref, m_ref):
    m_ref[...] = _elu(jnp.dot(g_ref[...], w2_ref[...],
                              preferred_element_type=jnp.float32) + b2_ref[...])


def _edge_mm(g, W2, b2):
    blk = 8000
    full = lambda s: pl.BlockSpec(s, lambda i: (0,) * len(s))
    return pl.pallas_call(
        _mm_body,
        grid=(E // blk,),
        in_specs=[
            pl.BlockSpec((blk, 64), lambda i: (i, 0)),
            full((64, 32)), full((1, 32)),
        ],
        out_specs=pl.BlockSpec((blk, 32), lambda i: (i, 0)),
        out_shape=jax.ShapeDtypeStruct((E, 32), jnp.float32),
    )(g, W2, b2)


def _final_body(h_ref, agg_ref, w1_ref, b1_ref, w2_ref, b2_ref, o_ref):
    h = h_ref[...] + agg_ref[...]
    t = _elu(jnp.dot(h, w1_ref[...], preferred_element_type=jnp.float32) + b1_ref[...])
    o = jnp.dot(t, w2_ref[...], preferred_element_type=jnp.float32) + b2_ref[...]
    o_ref[...] = jax.nn.sigmoid(o)


def _final(h, agg, W_out1, b_out1, W_out2, b_out2):
    blk = 5000
    full = lambda s: pl.BlockSpec(s, lambda i: (0,) * len(s))
    out2 = pl.pallas_call(
        _final_body,
        grid=(N // blk,),
        in_specs=[
            pl.BlockSpec((blk, 32), lambda i: (i, 0)),
            pl.BlockSpec((blk, 32), lambda i: (i, 0)),
            full((32, 16)), full((1, 16)), full((16, 1)), full((1, 1)),
        ],
        out_specs=pl.BlockSpec((blk, 1), lambda i: (i, 0)),
        out_shape=jax.ShapeDtypeStruct((N, 1), jnp.float32),
    )(h, agg, W_out1, b_out1, W_out2, b_out2)
    return out2[:, 0]


def _edge_layer(P, Q, src, dst, W2, b2):
    g = _elu(P[dst] + Q[src])
    m = _edge_mm(g, W2, b2)
    agg = jax.ops.segment_max(m, dst, num_segments=N)
    return jnp.where(jnp.isfinite(agg), agg, 0.0)


def kernel(x, edge_index, batch, emb_charge, emb_pdgid, emb_frompv, W_cat,
           b_cat, W_cont, b_cont, W_all, b_all, bn_gamma, bn_beta, W1_0, b1_0,
           W2_0, b2_0, W1_1, b1_1, W2_1, b2_1, W_out1, b_out1, W_out2, b_out2):
    W_cat_eff = jnp.concatenate([
        emb_charge @ W_cat[0:8],
        emb_pdgid @ W_cat[8:16],
        emb_frompv @ W_cat[16:24],
    ], axis=0)
    row = lambda b: b.reshape(1, -1)
    src, dst = edge_index[0], edge_index[1]

    h0, s1, s2 = _encode(x, W_cat_eff, row(b_cat), W_cont, row(b_cont),
                         W_all, row(b_all))

    WP0 = W1_0[:32] - W1_0[32:]
    WQ0 = W1_0[32:]
    h, P, Q = _pq_first(h0, s1, s2, row(bn_gamma), row(bn_beta), WP0, WQ0,
                        row(b1_0))
    agg0 = _edge_layer(P, Q, src, dst, W2_0, row(b2_0))

    WP1 = W1_1[:32] - W1_1[32:]
    WQ1 = W1_1[32:]
    h, P, Q = _pq_next(h, agg0, WP1, WQ1, row(b1_1))
    agg1 = _edge_layer(P, Q, src, dst, W2_1, row(b2_1))

    return _final(h, agg1, W_out1, row(b_out1), W_out2, row(b_out2))


# speedup vs baseline: 1.4628x; 1.4615x over previous
import functools

import jax
import jax.numpy as jnp
from jax import lax
from jax.experimental import pallas as pl
from jax.experimental.pallas import tpu as pltpu
from jax.experimental.pallas import tpu_sc as plsc

N = 50000
E = 800000
H = 32
MESG = 64
PDGS = (1, 2, 11, 13, 22, 130, 211)


def _elu(v):
    return jnp.where(v > 0, v, jnp.exp(jnp.minimum(v, 0.0)) - 1.0)


def _encode_body(x_ref, wcat_ref, bcat_ref, wcont_ref, bcont_ref,
                 wall_ref, ball_ref, h_ref, s1_ref, s2_ref):
    i = pl.program_id(0)
    x = x_ref[...]
    x_cont = x[:, :8]
    pdgv = jnp.abs(x[:, 8:9])
    chv = x[:, 9:10] + 1.0
    fpv = x[:, 10:11]
    oh_ch = jnp.concatenate(
        [(chv == float(v)).astype(jnp.float32) for v in range(3)], axis=1)
    oh_pdg = jnp.concatenate(
        [(pdgv == float(v)).astype(jnp.float32) for v in PDGS], axis=1)
    oh_fp = jnp.concatenate(
        [(fpv == float(v)).astype(jnp.float32) for v in range(4)], axis=1)
    oh = jnp.concatenate([oh_ch, oh_pdg, oh_fp], axis=1)
    emb_cat = _elu(jnp.dot(oh, wcat_ref[...], preferred_element_type=jnp.float32)
                   + bcat_ref[...])
    emb_cont = _elu(jnp.dot(x_cont, wcont_ref[...], preferred_element_type=jnp.float32)
                    + bcont_ref[...])
    hin = jnp.concatenate([emb_cat, emb_cont], axis=1)
    h = _elu(jnp.dot(hin, wall_ref[...], preferred_element_type=jnp.float32)
             + ball_ref[...])
    h_ref[...] = h

    @pl.when(i == 0)
    def _():
        s1_ref[...] = jnp.zeros_like(s1_ref)
        s2_ref[...] = jnp.zeros_like(s2_ref)

    s1_ref[...] += jnp.sum(h, axis=0, keepdims=True)
    s2_ref[...] += jnp.sum(h * h, axis=0, keepdims=True)


def _encode(x, W_cat_eff, b_cat, W_cont, b_cont, W_all, b_all):
    blk = 5000
    grid = N // blk
    full = lambda s: pl.BlockSpec(s, lambda i: (0,) * len(s))
    return pl.pallas_call(
        _encode_body,
        grid=(grid,),
        in_specs=[
            pl.BlockSpec((blk, 11), lambda i: (i, 0)),
            full((14, 16)), full((1, 16)), full((8, 16)), full((1, 16)),
            full((32, 32)), full((1, 32)),
        ],
        out_specs=[
            pl.BlockSpec((blk, 32), lambda i: (i, 0)),
            full((1, 32)), full((1, 32)),
        ],
        out_shape=[
            jax.ShapeDtypeStruct((N, 32), jnp.float32),
            jax.ShapeDtypeStruct((1, 32), jnp.float32),
            jax.ShapeDtypeStruct((1, 32), jnp.float32),
        ],
    )(x, W_cat_eff, b_cat, W_cont, b_cont, W_all, b_all)


def _pq_body(h_ref, s1_ref, s2_ref, g_ref, b_ref, wp_ref, wq_ref, bp_ref,
             h_out_ref, p_ref, q_ref):
    mean = s1_ref[...] / N
    var = s2_ref[...] / N - mean * mean
    inv = g_ref[...] * lax.rsqrt(var + 1e-5)
    h = h_ref[...] * inv + (b_ref[...] - mean * inv)
    h_out_ref[...] = h
    p_ref[...] = jnp.dot(h, wp_ref[...], preferred_element_type=jnp.float32) + bp_ref[...]
    q_ref[...] = jnp.dot(h, wq_ref[...], preferred_element_type=jnp.float32)


def _pq_first(h0, s1, s2, gamma, beta, WP, WQ, b1):
    blk = 5000
    full = lambda s: pl.BlockSpec(s, lambda i: (0,) * len(s))
    return pl.pallas_call(
        _pq_body,
        grid=(N // blk,),
        in_specs=[
            pl.BlockSpec((blk, 32), lambda i: (i, 0)),
            full((1, 32)), full((1, 32)), full((1, 32)), full((1, 32)),
            full((32, 64)), full((32, 64)), full((1, 64)),
        ],
        out_specs=[
            pl.BlockSpec((blk, 32), lambda i: (i, 0)),
            pl.BlockSpec((blk, 64), lambda i: (i, 0)),
            pl.BlockSpec((blk, 64), lambda i: (i, 0)),
        ],
        out_shape=[
            jax.ShapeDtypeStruct((N, 32), jnp.float32),
            jax.ShapeDtypeStruct((N, 64), jnp.float32),
            jax.ShapeDtypeStruct((N, 64), jnp.float32),
        ],
    )(h0, s1, s2, gamma, beta, WP, WQ, b1)


def _pq_next_body(h_ref, agg_ref, wp_ref, wq_ref, bp_ref,
                  h_out_ref, p_ref, q_ref):
    h = h_ref[...] + agg_ref[...]
    h_out_ref[...] = h
    p_ref[...] = jnp.dot(h, wp_ref[...], preferred_element_type=jnp.float32) + bp_ref[...]
    q_ref[...] = jnp.dot(h, wq_ref[...], preferred_element_type=jnp.float32)


def _pq_next(h, agg, WP, WQ, b1):
    blk = 5000
    full = lambda s: pl.BlockSpec(s, lambda i: (0,) * len(s))
    return pl.pallas_call(
        _pq_next_body,
        grid=(N // blk,),
        in_specs=[
            pl.BlockSpec((blk, 32), lambda i: (i, 0)),
            pl.BlockSpec((blk, 32), lambda i: (i, 0)),
            full((32, 64)), full((32, 64)), full((1, 64)),
        ],
        out_specs=[
            pl.BlockSpec((blk, 32), lambda i: (i, 0)),
            pl.BlockSpec((blk, 64), lambda i: (i, 0)),
            pl.BlockSpec((blk, 64), lambda i: (i, 0)),
        ],
        out_shape=[
            jax.ShapeDtypeStruct((N, 32), jnp.float32),
            jax.ShapeDtypeStruct((N, 64), jnp.float32),
            jax.ShapeDtypeStruct((N, 64), jnp.float32),
        ],
    )(h, agg, WP, WQ, b1)


def _mm_body(g_ref, w2_ref, b2_ref, m_ref):
    m_ref[...] = _elu(jnp.dot(g_ref[...], w2_ref[...],
                              preferred_element_type=jnp.float32) + b2_ref[...])


def _edge_mm(g, W2, b2):
    blk = 8000
    full = lambda s: pl.BlockSpec(s, lambda i: (0,) * len(s))
    return pl.pallas_call(
        _mm_body,
        grid=(E // blk,),
        in_specs=[
            pl.BlockSpec((blk, 64), lambda i: (i, 0)),
            full((64, 32)), full((1, 32)),
        ],
        out_specs=pl.BlockSpec((blk, 32), lambda i: (i, 0)),
        out_shape=jax.ShapeDtypeStruct((E, 32), jnp.float32),
    )(g, W2, b2)


def _final_body(h_ref, agg_ref, w1_ref, b1_ref, w2_ref, b2_ref, o_ref):
    h = h_ref[...] + agg_ref[...]
    t = _elu(jnp.dot(h, w1_ref[...], preferred_element_type=jnp.float32) + b1_ref[...])
    o = jnp.dot(t, w2_ref[...], preferred_element_type=jnp.float32) + b2_ref[...]
    o_ref[...] = jax.nn.sigmoid(o)


def _final(h, agg, W_out1, b_out1, W_out2, b_out2):
    blk = 5000
    full = lambda s: pl.BlockSpec(s, lambda i: (0,) * len(s))
    out2 = pl.pallas_call(
        _final_body,
        grid=(N // blk,),
        in_specs=[
            pl.BlockSpec((blk, 32), lambda i: (i, 0)),
            pl.BlockSpec((blk, 32), lambda i: (i, 0)),
            full((32, 16)), full((1, 16)), full((16, 1)), full((1, 1)),
        ],
        out_specs=pl.BlockSpec((blk, 1), lambda i: (i, 0)),
        out_shape=jax.ShapeDtypeStruct((N, 1), jnp.float32),
    )(h, agg, W_out1, b_out1, W_out2, b_out2)
    return out2[:, 0]


SC_NC, SC_NS = 2, 16
SC_NW = SC_NC * SC_NS
EPW = E // SC_NW
EP2 = E
GW = 200


def _sc_elu(v):
    return jnp.where(v > 0, v, jnp.exp(jnp.minimum(v, 0.0)) - 1.0)


def _gather_body(p_hbm, q_hbm, src_hbm, dst_hbm, out_hbm,
                 idx_s, idx_d, rows_p, rows_q, sem_p, sem_q):
    wid = lax.axis_index("s") * SC_NC + lax.axis_index("c")
    base = wid * EPW

    def window(w, carry):
        b = base + w * GW
        pltpu.sync_copy(src_hbm.at[pl.ds(b, GW)], idx_s)
        pltpu.sync_copy(dst_hbm.at[pl.ds(b, GW)], idx_d)
        cp_q = pltpu.async_copy(q_hbm.at[idx_s], rows_q, sem_q)
        cp_p = pltpu.async_copy(p_hbm.at[idx_d], rows_p, sem_p)
        cp_q.wait()
        cp_p.wait()

        def edge(e, c):
            for j in range(4):
                v = rows_p[e, pl.ds(j * 16, 16)] + rows_q[e, pl.ds(j * 16, 16)]
                rows_p[e, pl.ds(j * 16, 16)] = _sc_elu(v)
            return c

        lax.fori_loop(0, GW, edge, 0, unroll=2)
        pltpu.sync_copy(rows_p, out_hbm.at[pl.ds(b, GW)])
        return carry

    lax.fori_loop(0, EPW // GW, window, 0)


def _sc_gather(P, Q, src, dst):
    mesh = plsc.VectorSubcoreMesh(core_axis_name="c", subcore_axis_name="s",
                                  num_cores=SC_NC, num_subcores=SC_NS)
    return pl.kernel(
        _gather_body,
        out_type=jax.ShapeDtypeStruct((E, 64), jnp.float32),
        mesh=mesh,
        compiler_params=pltpu.CompilerParams(use_tc_tiling_on_sc=False),
        scratch_types=[
            pltpu.VMEM((GW,), jnp.int32),
            pltpu.VMEM((GW,), jnp.int32),
            pltpu.VMEM((GW, 64), jnp.float32),
            pltpu.VMEM((GW, 64), jnp.float32),
            pltpu.SemaphoreType.DMA,
            pltpu.SemaphoreType.DMA,
        ],
    )(P, Q, src, dst)


def _edge_layer(P, Q, src, dst, W2, b2):
    g = _sc_gather(P, Q, src, dst)
    m = _edge_mm(g, W2, b2)
    agg = jax.ops.segment_max(m, dst, num_segments=N)
    return jnp.where(jnp.isfinite(agg), agg, 0.0)


def kernel(x, edge_index, batch, emb_charge, emb_pdgid, emb_frompv, W_cat,
           b_cat, W_cont, b_cont, W_all, b_all, bn_gamma, bn_beta, W1_0, b1_0,
           W2_0, b2_0, W1_1, b1_1, W2_1, b2_1, W_out1, b_out1, W_out2, b_out2):
    W_cat_eff = jnp.concatenate([
        emb_charge @ W_cat[0:8],
        emb_pdgid @ W_cat[8:16],
        emb_frompv @ W_cat[16:24],
    ], axis=0)
    row = lambda b: b.reshape(1, -1)
    src, dst = edge_index[0], edge_index[1]

    h0, s1, s2 = _encode(x, W_cat_eff, row(b_cat), W_cont, row(b_cont),
                         W_all, row(b_all))

    WP0 = W1_0[:32] - W1_0[32:]
    WQ0 = W1_0[32:]
    h, P, Q = _pq_first(h0, s1, s2, row(bn_gamma), row(bn_beta), WP0, WQ0,
                        row(b1_0))
    agg0 = _edge_layer(P, Q, src, dst, W2_0, row(b2_0))

    WP1 = W1_1[:32] - W1_1[32:]
    WQ1 = W1_1[32:]
    h, P, Q = _pq_next(h, agg0, WP1, WQ1, row(b1_1))
    agg1 = _edge_layer(P, Q, src, dst, W2_1, row(b2_1))

    return _final(h, agg1, W_out1, row(b_out1), W_out2, row(b_out2))


# speedup vs baseline: 1.9552x; 1.3366x over previous
import functools

import jax
import jax.numpy as jnp
from jax import lax
from jax.experimental import pallas as pl
from jax.experimental.pallas import tpu as pltpu
from jax.experimental.pallas import tpu_sc as plsc

N = 50000
E = 800000
H = 32
MESG = 64
PDGS = (1, 2, 11, 13, 22, 130, 211)


def _elu(v):
    return jnp.where(v > 0, v, jnp.exp(jnp.minimum(v, 0.0)) - 1.0)


def _encode_body(x_ref, wcat_ref, bcat_ref, wcont_ref, bcont_ref,
                 wall_ref, ball_ref, h_ref, s1_ref, s2_ref):
    i = pl.program_id(0)
    x = x_ref[...]
    x_cont = x[:, :8]
    pdgv = jnp.abs(x[:, 8:9])
    chv = x[:, 9:10] + 1.0
    fpv = x[:, 10:11]
    oh_ch = jnp.concatenate(
        [(chv == float(v)).astype(jnp.float32) for v in range(3)], axis=1)
    oh_pdg = jnp.concatenate(
        [(pdgv == float(v)).astype(jnp.float32) for v in PDGS], axis=1)
    oh_fp = jnp.concatenate(
        [(fpv == float(v)).astype(jnp.float32) for v in range(4)], axis=1)
    oh = jnp.concatenate([oh_ch, oh_pdg, oh_fp], axis=1)
    emb_cat = _elu(jnp.dot(oh, wcat_ref[...], preferred_element_type=jnp.float32)
                   + bcat_ref[...])
    emb_cont = _elu(jnp.dot(x_cont, wcont_ref[...], preferred_element_type=jnp.float32)
                    + bcont_ref[...])
    hin = jnp.concatenate([emb_cat, emb_cont], axis=1)
    h = _elu(jnp.dot(hin, wall_ref[...], preferred_element_type=jnp.float32)
             + ball_ref[...])
    h_ref[...] = h

    @pl.when(i == 0)
    def _():
        s1_ref[...] = jnp.zeros_like(s1_ref)
        s2_ref[...] = jnp.zeros_like(s2_ref)

    s1_ref[...] += jnp.sum(h, axis=0, keepdims=True)
    s2_ref[...] += jnp.sum(h * h, axis=0, keepdims=True)


def _encode(x, W_cat_eff, b_cat, W_cont, b_cont, W_all, b_all):
    blk = 5000
    grid = N // blk
    full = lambda s: pl.BlockSpec(s, lambda i: (0,) * len(s))
    return pl.pallas_call(
        _encode_body,
        grid=(grid,),
        in_specs=[
            pl.BlockSpec((blk, 11), lambda i: (i, 0)),
            full((14, 16)), full((1, 16)), full((8, 16)), full((1, 16)),
            full((32, 32)), full((1, 32)),
        ],
        out_specs=[
            pl.BlockSpec((blk, 32), lambda i: (i, 0)),
            full((1, 32)), full((1, 32)),
        ],
        out_shape=[
            jax.ShapeDtypeStruct((N, 32), jnp.float32),
            jax.ShapeDtypeStruct((1, 32), jnp.float32),
            jax.ShapeDtypeStruct((1, 32), jnp.float32),
        ],
    )(x, W_cat_eff, b_cat, W_cont, b_cont, W_all, b_all)


def _pq_body(h_ref, s1_ref, s2_ref, g_ref, b_ref, wp_ref, wq_ref, bp_ref,
             h_out_ref, p_ref, q_ref):
    mean = s1_ref[...] / N
    var = s2_ref[...] / N - mean * mean
    inv = g_ref[...] * lax.rsqrt(var + 1e-5)
    h = h_ref[...] * inv + (b_ref[...] - mean * inv)
    h_out_ref[...] = h
    p_ref[...] = jnp.dot(h, wp_ref[...], preferred_element_type=jnp.float32) + bp_ref[...]
    q_ref[...] = jnp.dot(h, wq_ref[...], preferred_element_type=jnp.float32)


def _pq_first(h0, s1, s2, gamma, beta, WP, WQ, b1):
    blk = 5000
    full = lambda s: pl.BlockSpec(s, lambda i: (0,) * len(s))
    return pl.pallas_call(
        _pq_body,
        grid=(N // blk,),
        in_specs=[
            pl.BlockSpec((blk, 32), lambda i: (i, 0)),
            full((1, 32)), full((1, 32)), full((1, 32)), full((1, 32)),
            full((32, 64)), full((32, 64)), full((1, 64)),
        ],
        out_specs=[
            pl.BlockSpec((blk, 32), lambda i: (i, 0)),
            pl.BlockSpec((blk, 64), lambda i: (i, 0)),
            pl.BlockSpec((blk, 64), lambda i: (i, 0)),
        ],
        out_shape=[
            jax.ShapeDtypeStruct((N, 32), jnp.float32),
            jax.ShapeDtypeStruct((N, 64), jnp.float32),
            jax.ShapeDtypeStruct((N, 64), jnp.float32),
        ],
    )(h0, s1, s2, gamma, beta, WP, WQ, b1)


def _pq_next_body(h_ref, agg_ref, wp_ref, wq_ref, bp_ref,
                  h_out_ref, p_ref, q_ref):
    h = h_ref[...] + agg_ref[...]
    h_out_ref[...] = h
    p_ref[...] = jnp.dot(h, wp_ref[...], preferred_element_type=jnp.float32) + bp_ref[...]
    q_ref[...] = jnp.dot(h, wq_ref[...], preferred_element_type=jnp.float32)


def _pq_next(h, agg, WP, WQ, b1):
    blk = 5000
    full = lambda s: pl.BlockSpec(s, lambda i: (0,) * len(s))
    return pl.pallas_call(
        _pq_next_body,
        grid=(N // blk,),
        in_specs=[
            pl.BlockSpec((blk, 32), lambda i: (i, 0)),
            pl.BlockSpec((blk, 32), lambda i: (i, 0)),
            full((32, 64)), full((32, 64)), full((1, 64)),
        ],
        out_specs=[
            pl.BlockSpec((blk, 32), lambda i: (i, 0)),
            pl.BlockSpec((blk, 64), lambda i: (i, 0)),
            pl.BlockSpec((blk, 64), lambda i: (i, 0)),
        ],
        out_shape=[
            jax.ShapeDtypeStruct((N, 32), jnp.float32),
            jax.ShapeDtypeStruct((N, 64), jnp.float32),
            jax.ShapeDtypeStruct((N, 64), jnp.float32),
        ],
    )(h, agg, WP, WQ, b1)


def _mm_body(g_ref, w2_ref, b2_ref, m_ref):
    g = _elu(g_ref[...])
    m_ref[...] = _elu(jnp.dot(g, w2_ref[...],
                              preferred_element_type=jnp.float32) + b2_ref[...])


def _edge_mm(g, W2, b2):
    blk = 8000
    full = lambda s: pl.BlockSpec(s, lambda i: (0,) * len(s))
    return pl.pallas_call(
        _mm_body,
        grid=(E // blk,),
        in_specs=[
            pl.BlockSpec((blk, 64), lambda i: (i, 0)),
            full((64, 32)), full((1, 32)),
        ],
        out_specs=pl.BlockSpec((blk, 32), lambda i: (i, 0)),
        out_shape=jax.ShapeDtypeStruct((E, 32), jnp.float32),
    )(g, W2, b2)


def _final_body(h_ref, agg_ref, w1_ref, b1_ref, w2_ref, b2_ref, o_ref):
    h = h_ref[...] + agg_ref[...]
    t = _elu(jnp.dot(h, w1_ref[...], preferred_element_type=jnp.float32) + b1_ref[...])
    o = jnp.dot(t, w2_ref[...], preferred_element_type=jnp.float32) + b2_ref[...]
    o_ref[...] = jax.nn.sigmoid(o)


def _final(h, agg, W_out1, b_out1, W_out2, b_out2):
    blk = 5000
    full = lambda s: pl.BlockSpec(s, lambda i: (0,) * len(s))
    out2 = pl.pallas_call(
        _final_body,
        grid=(N // blk,),
        in_specs=[
            pl.BlockSpec((blk, 32), lambda i: (i, 0)),
            pl.BlockSpec((blk, 32), lambda i: (i, 0)),
            full((32, 16)), full((1, 16)), full((16, 1)), full((1, 1)),
        ],
        out_specs=pl.BlockSpec((blk, 1), lambda i: (i, 0)),
        out_shape=jax.ShapeDtypeStruct((N, 1), jnp.float32),
    )(h, agg, W_out1, b_out1, W_out2, b_out2)
    return out2[:, 0]


SC_NC, SC_NS = 2, 16
SC_NW = SC_NC * SC_NS
EPW = E // SC_NW
EP2 = E
GW = 200


def _sc_elu(v):
    return jnp.where(v > 0, v, jnp.exp(jnp.minimum(v, 0.0)) - 1.0)


NWIN = EPW // GW


def _gather_body(p_hbm, q_hbm, src_hbm, dst_hbm, out_hbm,
                 idx_s, idx_d, rows_p, rows_q, g_buf,
                 sem_i0, sem_i1, sem_p0, sem_p1, sem_q0, sem_q1,
                 sem_o0, sem_o1):
    wid = lax.axis_index("s") * SC_NC + lax.axis_index("c")
    base = wid * EPW
    sem_i = (sem_i0, sem_i1)
    sem_p = (sem_p0, sem_p1)
    sem_q = (sem_q0, sem_q1)
    sem_o = (sem_o0, sem_o1)

    def start_idx(w, s):
        b = base + w * GW
        pltpu.async_copy(src_hbm.at[pl.ds(b, GW)], idx_s.at[s], sem_i[s])
        pltpu.async_copy(dst_hbm.at[pl.ds(b, GW)], idx_d.at[s], sem_i[s])

    def wait_idx(s):
        pltpu.make_async_copy(src_hbm.at[pl.ds(base, GW)], idx_s.at[s],
                              sem_i[s]).wait()
        pltpu.make_async_copy(dst_hbm.at[pl.ds(base, GW)], idx_d.at[s],
                              sem_i[s]).wait()

    def start_rows(s):
        pltpu.async_copy(q_hbm.at[idx_s.at[s]], rows_q.at[s], sem_q[s])
        pltpu.async_copy(p_hbm.at[idx_d.at[s]], rows_p.at[s], sem_p[s])

    def wait_rows(s):
        pltpu.make_async_copy(q_hbm.at[idx_s.at[s]], rows_q.at[s],
                              sem_q[s]).wait()
        pltpu.make_async_copy(p_hbm.at[idx_d.at[s]], rows_p.at[s],
                              sem_p[s]).wait()

    def start_out(w, s):
        b = base + w * GW
        pltpu.async_copy(g_buf.at[s], out_hbm.at[pl.ds(b, GW)], sem_o[s])

    def wait_out(s):
        pltpu.make_async_copy(g_buf.at[s], out_hbm.at[pl.ds(base, GW)],
                              sem_o[s]).wait()

    def compute(s):
        def edge(e, c):
            for j in range(4):
                sl = pl.ds(j * 16, 16)
                g_buf[s, e, sl] = rows_p[s, e, sl] + rows_q[s, e, sl]
            return c

        lax.fori_loop(0, GW, edge, 0, unroll=4)

    start_idx(0, 0)
    wait_idx(0)
    start_rows(0)
    start_idx(1, 1)

    def half(w, s, o):
        @pl.when(w < NWIN)
        def _():
            wait_rows(s)

            @pl.when(w + 2 < NWIN)
            def _():
                start_idx(w + 2, s)

            @pl.when(w + 1 < NWIN)
            def _():
                wait_idx(o)
                start_rows(o)

            @pl.when(w >= 2)
            def _():
                wait_out(s)

            compute(s)
            start_out(w, s)

    def body(i, c):
        half(2 * i, 0, 1)
        half(2 * i + 1, 1, 0)
        return c

    lax.fori_loop(0, (NWIN + 1) // 2, body, 0)
    wait_out(0)
    wait_out(1)


def _sc_gather(P, Q, src, dst):
    mesh = plsc.VectorSubcoreMesh(core_axis_name="c", subcore_axis_name="s",
                                  num_cores=SC_NC, num_subcores=SC_NS)
    return pl.kernel(
        _gather_body,
        out_type=jax.ShapeDtypeStruct((E, 64), jnp.float32),
        mesh=mesh,
        compiler_params=pltpu.CompilerParams(use_tc_tiling_on_sc=False),
        scratch_types=[
            pltpu.VMEM((2, GW), jnp.int32),
            pltpu.VMEM((2, GW), jnp.int32),
            pltpu.VMEM((2, GW, 64), jnp.float32),
            pltpu.VMEM((2, GW, 64), jnp.float32),
            pltpu.VMEM((2, GW, 64), jnp.float32),
        ] + [pltpu.SemaphoreType.DMA] * 8,
    )(P, Q, src, dst)


def _edge_layer(P, Q, src, dst, W2, b2):
    g = _sc_gather(P, Q, src, dst)
    m = _edge_mm(g, W2, b2)
    agg = jax.ops.segment_max(m, dst, num_segments=N)
    return jnp.where(jnp.isfinite(agg), agg, 0.0)


def kernel(x, edge_index, batch, emb_charge, emb_pdgid, emb_frompv, W_cat,
           b_cat, W_cont, b_cont, W_all, b_all, bn_gamma, bn_beta, W1_0, b1_0,
           W2_0, b2_0, W1_1, b1_1, W2_1, b2_1, W_out1, b_out1, W_out2, b_out2):
    W_cat_eff = jnp.concatenate([
        emb_charge @ W_cat[0:8],
        emb_pdgid @ W_cat[8:16],
        emb_frompv @ W_cat[16:24],
    ], axis=0)
    row = lambda b: b.reshape(1, -1)
    src, dst = edge_index[0], edge_index[1]

    h0, s1, s2 = _encode(x, W_cat_eff, row(b_cat), W_cont, row(b_cont),
                         W_all, row(b_all))

    WP0 = W1_0[:32] - W1_0[32:]
    WQ0 = W1_0[32:]
    h, P, Q = _pq_first(h0, s1, s2, row(bn_gamma), row(bn_beta), WP0, WQ0,
                        row(b1_0))
    agg0 = _edge_layer(P, Q, src, dst, W2_0, row(b2_0))

    WP1 = W1_1[:32] - W1_1[32:]
    WQ1 = W1_1[32:]
    h, P, Q = _pq_next(h, agg0, WP1, WQ1, row(b1_1))
    agg1 = _edge_layer(P, Q, src, dst, W2_1, row(b2_1))

    return _final(h, agg1, W_out1, row(b_out1), W_out2, row(b_out2))


# speedup vs baseline: 2.2286x; 1.1398x over previous
import functools

import jax
import jax.numpy as jnp
from jax import lax
from jax.experimental import pallas as pl
from jax.experimental.pallas import tpu as pltpu
from jax.experimental.pallas import tpu_sc as plsc

N = 50000
E = 800000
H = 32
MESG = 64
PDGS = (1, 2, 11, 13, 22, 130, 211)


def _elu(v):
    return jnp.where(v > 0, v, jnp.exp(jnp.minimum(v, 0.0)) - 1.0)


def _encode_body(x_ref, wcat_ref, bcat_ref, wcont_ref, bcont_ref,
                 wall_ref, ball_ref, h_ref, s1_ref, s2_ref):
    i = pl.program_id(0)
    x = x_ref[...]
    x_cont = x[:, :8]
    pdgv = jnp.abs(x[:, 8:9])
    chv = x[:, 9:10] + 1.0
    fpv = x[:, 10:11]
    oh_ch = jnp.concatenate(
        [(chv == float(v)).astype(jnp.float32) for v in range(3)], axis=1)
    oh_pdg = jnp.concatenate(
        [(pdgv == float(v)).astype(jnp.float32) for v in PDGS], axis=1)
    oh_fp = jnp.concatenate(
        [(fpv == float(v)).astype(jnp.float32) for v in range(4)], axis=1)
    oh = jnp.concatenate([oh_ch, oh_pdg, oh_fp], axis=1)
    emb_cat = _elu(jnp.dot(oh, wcat_ref[...], preferred_element_type=jnp.float32)
                   + bcat_ref[...])
    emb_cont = _elu(jnp.dot(x_cont, wcont_ref[...], preferred_element_type=jnp.float32)
                    + bcont_ref[...])
    hin = jnp.concatenate([emb_cat, emb_cont], axis=1)
    h = _elu(jnp.dot(hin, wall_ref[...], preferred_element_type=jnp.float32)
             + ball_ref[...])
    h_ref[...] = h

    @pl.when(i == 0)
    def _():
        s1_ref[...] = jnp.zeros_like(s1_ref)
        s2_ref[...] = jnp.zeros_like(s2_ref)

    s1_ref[...] += jnp.sum(h, axis=0, keepdims=True)
    s2_ref[...] += jnp.sum(h * h, axis=0, keepdims=True)


def _encode(x, W_cat_eff, b_cat, W_cont, b_cont, W_all, b_all):
    blk = 5000
    grid = N // blk
    full = lambda s: pl.BlockSpec(s, lambda i: (0,) * len(s))
    return pl.pallas_call(
        _encode_body,
        grid=(grid,),
        in_specs=[
            pl.BlockSpec((blk, 11), lambda i: (i, 0)),
            full((14, 16)), full((1, 16)), full((8, 16)), full((1, 16)),
            full((32, 32)), full((1, 32)),
        ],
        out_specs=[
            pl.BlockSpec((blk, 32), lambda i: (i, 0)),
            full((1, 32)), full((1, 32)),
        ],
        out_shape=[
            jax.ShapeDtypeStruct((N, 32), jnp.float32),
            jax.ShapeDtypeStruct((1, 32), jnp.float32),
            jax.ShapeDtypeStruct((1, 32), jnp.float32),
        ],
    )(x, W_cat_eff, b_cat, W_cont, b_cont, W_all, b_all)


def _pq_body(h_ref, s1_ref, s2_ref, g_ref, b_ref, wp_ref, wq_ref, bp_ref,
             h_out_ref, p_ref, q_ref):
    mean = s1_ref[...] / N
    var = s2_ref[...] / N - mean * mean
    inv = g_ref[...] * lax.rsqrt(var + 1e-5)
    h = h_ref[...] * inv + (b_ref[...] - mean * inv)
    h_out_ref[...] = h
    p_ref[...] = jnp.dot(h, wp_ref[...], preferred_element_type=jnp.float32) + bp_ref[...]
    q_ref[...] = jnp.dot(h, wq_ref[...], preferred_element_type=jnp.float32)


def _pq_first(h0, s1, s2, gamma, beta, WP, WQ, b1):
    blk = 5000
    full = lambda s: pl.BlockSpec(s, lambda i: (0,) * len(s))
    return pl.pallas_call(
        _pq_body,
        grid=(N // blk,),
        in_specs=[
            pl.BlockSpec((blk, 32), lambda i: (i, 0)),
            full((1, 32)), full((1, 32)), full((1, 32)), full((1, 32)),
            full((32, 64)), full((32, 64)), full((1, 64)),
        ],
        out_specs=[
            pl.BlockSpec((blk, 32), lambda i: (i, 0)),
            pl.BlockSpec((blk, 64), lambda i: (i, 0)),
            pl.BlockSpec((blk, 64), lambda i: (i, 0)),
        ],
        out_shape=[
            jax.ShapeDtypeStruct((N, 32), jnp.float32),
            jax.ShapeDtypeStruct((N, 64), jnp.float32),
            jax.ShapeDtypeStruct((N, 64), jnp.float32),
        ],
    )(h0, s1, s2, gamma, beta, WP, WQ, b1)


def _pq_next_body(h_ref, agg_ref, wp_ref, wq_ref, bp_ref,
                  h_out_ref, p_ref, q_ref):
    h = h_ref[...] + agg_ref[...]
    h_out_ref[...] = h
    p_ref[...] = jnp.dot(h, wp_ref[...], preferred_element_type=jnp.float32) + bp_ref[...]
    q_ref[...] = jnp.dot(h, wq_ref[...], preferred_element_type=jnp.float32)


def _pq_next(h, agg, WP, WQ, b1):
    blk = 5000
    full = lambda s: pl.BlockSpec(s, lambda i: (0,) * len(s))
    return pl.pallas_call(
        _pq_next_body,
        grid=(N // blk,),
        in_specs=[
            pl.BlockSpec((blk, 32), lambda i: (i, 0)),
            pl.BlockSpec((blk, 32), lambda i: (i, 0)),
            full((32, 64)), full((32, 64)), full((1, 64)),
        ],
        out_specs=[
            pl.BlockSpec((blk, 32), lambda i: (i, 0)),
            pl.BlockSpec((blk, 64), lambda i: (i, 0)),
            pl.BlockSpec((blk, 64), lambda i: (i, 0)),
        ],
        out_shape=[
            jax.ShapeDtypeStruct((N, 32), jnp.float32),
            jax.ShapeDtypeStruct((N, 64), jnp.float32),
            jax.ShapeDtypeStruct((N, 64), jnp.float32),
        ],
    )(h, agg, WP, WQ, b1)


def _mm_body(g_ref, w2_ref, b2_ref, m_ref):
    g = _elu(g_ref[...])
    m_ref[...] = _elu(jnp.dot(g, w2_ref[...],
                              preferred_element_type=jnp.float32) + b2_ref[...])


def _edge_mm(g, W2, b2):
    blk = 6400
    full = lambda s: pl.BlockSpec(s, lambda i: (0,) * len(s))
    return pl.pallas_call(
        _mm_body,
        grid=(EP2 // blk,),
        in_specs=[
            pl.BlockSpec((blk, 64), lambda i: (i, 0)),
            full((64, 32)), full((1, 32)),
        ],
        out_specs=pl.BlockSpec((blk, 32), lambda i: (i, 0)),
        out_shape=jax.ShapeDtypeStruct((EP2, 32), jnp.float32),
    )(g, W2, b2)


def _final_body(h_ref, agg_ref, w1_ref, b1_ref, w2_ref, b2_ref, o_ref):
    h = h_ref[...] + agg_ref[...]
    t = _elu(jnp.dot(h, w1_ref[...], preferred_element_type=jnp.float32) + b1_ref[...])
    o = jnp.dot(t, w2_ref[...], preferred_element_type=jnp.float32) + b2_ref[...]
    o_ref[...] = jax.nn.sigmoid(o)


def _final(h, agg, W_out1, b_out1, W_out2, b_out2):
    blk = 5000
    full = lambda s: pl.BlockSpec(s, lambda i: (0,) * len(s))
    out2 = pl.pallas_call(
        _final_body,
        grid=(N // blk,),
        in_specs=[
            pl.BlockSpec((blk, 32), lambda i: (i, 0)),
            pl.BlockSpec((blk, 32), lambda i: (i, 0)),
            full((32, 16)), full((1, 16)), full((16, 1)), full((1, 1)),
        ],
        out_specs=pl.BlockSpec((blk, 1), lambda i: (i, 0)),
        out_shape=jax.ShapeDtypeStruct((N, 1), jnp.float32),
    )(h, agg, W_out1, b_out1, W_out2, b_out2)
    return out2[:, 0]


SC_NC, SC_NS = 2, 16
SC_NW = SC_NC * SC_NS
C = E // SC_NW
CP = 25600
EPAD = SC_NW * CP
EP2 = EPAD + 6400
EPW = CP
GW = 200
NWIN = EPW // GW
NBIN = 32
BINW = 1563
NPAD = NBIN * BINW
WS = 256
SENT = -3.0e38


def _sc_mesh():
    return plsc.VectorSubcoreMesh(core_axis_name="c", subcore_axis_name="s",
                                  num_cores=SC_NC, num_subcores=SC_NS)


def _wid():
    return lax.axis_index("s") * SC_NC + lax.axis_index("c")


def _iota16():
    return lax.iota(jnp.int32, 16)


def _bin_of(n):
    return lax.shift_right_logical(n * 42936, 26)


def _gather_body(p_hbm, q_hbm, src_hbm, dst_hbm, out_hbm,
                 idx_s, idx_d, rows_p, rows_q, g_buf,
                 sem_i0, sem_i1, sem_p0, sem_p1, sem_q0, sem_q1,
                 sem_o0, sem_o1):
    wid = lax.axis_index("s") * SC_NC + lax.axis_index("c")
    base = wid * EPW
    sem_i = (sem_i0, sem_i1)
    sem_p = (sem_p0, sem_p1)
    sem_q = (sem_q0, sem_q1)
    sem_o = (sem_o0, sem_o1)

    def start_idx(w, s):
        b = base + w * GW
        pltpu.async_copy(src_hbm.at[pl.ds(b, GW)], idx_s.at[s], sem_i[s])
        pltpu.async_copy(dst_hbm.at[pl.ds(b, GW)], idx_d.at[s], sem_i[s])

    def wait_idx(s):
        pltpu.make_async_copy(src_hbm.at[pl.ds(base, GW)], idx_s.at[s],
                              sem_i[s]).wait()
        pltpu.make_async_copy(dst_hbm.at[pl.ds(base, GW)], idx_d.at[s],
                              sem_i[s]).wait()

    def start_rows(s):
        pltpu.async_copy(q_hbm.at[idx_s.at[s]], rows_q.at[s], sem_q[s])
        pltpu.async_copy(p_hbm.at[idx_d.at[s]], rows_p.at[s], sem_p[s])

    def wait_rows(s):
        pltpu.make_async_copy(q_hbm.at[idx_s.at[s]], rows_q.at[s],
                              sem_q[s]).wait()
        pltpu.make_async_copy(p_hbm.at[idx_d.at[s]], rows_p.at[s],
                              sem_p[s]).wait()

    def start_out(w, s):
        b = base + w * GW
        pltpu.async_copy(g_buf.at[s], out_hbm.at[pl.ds(b, GW)], sem_o[s])

    def wait_out(s):
        pltpu.make_async_copy(g_buf.at[s], out_hbm.at[pl.ds(base, GW)],
                              sem_o[s]).wait()

    def compute(s):
        def edge(e, c):
            for j in range(4):
                sl = pl.ds(j * 16, 16)
                g_buf[s, e, sl] = rows_p[s, e, sl] + rows_q[s, e, sl]
            return c

        lax.fori_loop(0, GW, edge, 0, unroll=4)

    start_idx(0, 0)
    wait_idx(0)
    start_rows(0)
    start_idx(1, 1)

    def half(w, s, o):
        @pl.when(w < NWIN)
        def _():
            wait_rows(s)

            @pl.when(w + 2 < NWIN)
            def _():
                start_idx(w + 2, s)

            @pl.when(w + 1 < NWIN)
            def _():
                wait_idx(o)
                start_rows(o)

            @pl.when(w >= 2)
            def _():
                wait_out(s)

            compute(s)
            start_out(w, s)

    def body(i, c):
        half(2 * i, 0, 1)
        half(2 * i + 1, 1, 0)
        return c

    lax.fori_loop(0, (NWIN + 1) // 2, body, 0)
    wait_out(0)
    wait_out(1)


def _sc_gather(P, Q, src, dst):
    mesh = plsc.VectorSubcoreMesh(core_axis_name="c", subcore_axis_name="s",
                                  num_cores=SC_NC, num_subcores=SC_NS)
    return pl.kernel(
        _gather_body,
        out_type=jax.ShapeDtypeStruct((EP2, 64), jnp.float32),
        mesh=mesh,
        compiler_params=pltpu.CompilerParams(use_tc_tiling_on_sc=False),
        scratch_types=[
            pltpu.VMEM((2, GW), jnp.int32),
            pltpu.VMEM((2, GW), jnp.int32),
            pltpu.VMEM((2, GW, 64), jnp.float32),
            pltpu.VMEM((2, GW, 64), jnp.float32),
            pltpu.VMEM((2, GW, 64), jnp.float32),
        ] + [pltpu.SemaphoreType.DMA] * 8,
    )(P, Q, src, dst)


_BIN_WINS = ((0, 6400), (6400, 6400), (12800, 6400), (19200, 5800))


def _runs(s):
    i16 = _iota16()
    prev = s[jnp.maximum(i16 - 1, 0)]
    first = (i16 == 0) | (s != prev)
    nxt = s[jnp.minimum(i16 + 1, 15)]
    last = (i16 == 15) | (s != nxt)
    pstart = plsc.cummax(jnp.where(first, i16, 0))
    rank = i16 - pstart
    return first, last, rank


def _bin_body(src_hbm, dst_hbm, srcp_hbm, dstp_hbm, offs_hbm, cnts_hbm,
              dstw, srcw, out_src, out_dst, cnt_ref, cur_ref, obuf, cbuf):
    cw = _wid()
    i16 = _iota16()
    zero16 = i16 * 0
    cbase_raw = cw * C
    cbase_p = cw * CP

    cnt_ref[pl.ds(0, 16)] = zero16
    cnt_ref[pl.ds(16, 16)] = zero16
    cnt_ref[pl.ds(32, 16)] = zero16

    def pre(v, c):
        out_src[pl.ds(v * 16, 16)] = zero16
        out_dst[pl.ds(v * 16, 16)] = zero16
        return c

    lax.fori_loop(0, CP // 16, pre, 0)

    for wb, wlen in _BIN_WINS:
        pltpu.sync_copy(dst_hbm.at[pl.ds(cbase_raw + wb, wlen)],
                        dstw.at[pl.ds(0, wlen)])
        nv = (wlen + 15) // 16

        def cvec(v, c, wb=wb):
            d = dstw[pl.ds(v * 16, 16)]
            b = jnp.where(i16 < (C - wb - v * 16), _bin_of(d), NBIN)
            s, _ = plsc.sort_key_val(b, i16)
            _, last, rank = _runs(s)
            cur = plsc.load_gather(cnt_ref, [s])
            plsc.store_scatter(cnt_ref, [s], cur + rank + 1, mask=last)
            return c

        lax.fori_loop(0, nv, cvec, 0)

    cnt0 = cnt_ref[pl.ds(0, 16)]
    cnt1 = cnt_ref[pl.ds(16, 16)]
    r0 = jnp.bitwise_and(cnt0 + 7, -8)
    r1 = jnp.bitwise_and(cnt1 + 7, -8)
    c0 = plsc.cumsum(r0)
    c1 = plsc.cumsum(r1)
    tot0 = c0[zero16 + 15]
    excl0 = c0 - r0
    excl1 = c1 - r1 + tot0
    end_all = c1[zero16 + 15] + tot0
    cur_ref[pl.ds(0, 16)] = excl0
    cur_ref[pl.ds(16, 16)] = excl1
    cur_ref[pl.ds(32, 16)] = end_all + zero16
    obuf[pl.ds(0, 16)] = excl0 + cbase_p
    obuf[pl.ds(16, 16)] = excl1 + cbase_p
    cbuf[pl.ds(0, 16)] = cnt0
    cbuf[pl.ds(16, 16)] = cnt1
    pltpu.sync_copy(obuf, offs_hbm.at[cw])
    pltpu.sync_copy(cbuf, cnts_hbm.at[cw])

    for wb, wlen in _BIN_WINS:
        pltpu.sync_copy(dst_hbm.at[pl.ds(cbase_raw + wb, wlen)],
                        dstw.at[pl.ds(0, wlen)])
        pltpu.sync_copy(src_hbm.at[pl.ds(cbase_raw + wb, wlen)],
                        srcw.at[pl.ds(0, wlen)])
        nv = (wlen + 15) // 16

        def pvec(v, c, wb=wb):
            d = dstw[pl.ds(v * 16, 16)]
            sv = srcw[pl.ds(v * 16, 16)]
            valid = i16 < (C - wb - v * 16)
            b = jnp.where(valid, _bin_of(d), NBIN)
            s, perm = plsc.sort_key_val(b, i16)
            _, last, rank = _runs(s)
            cur = plsc.load_gather(cur_ref, [s])
            pos = cur + rank
            plsc.store_scatter(out_dst, [pos], jnp.where(valid, d, 0)[perm])
            plsc.store_scatter(out_src, [pos], jnp.where(valid, sv, 0)[perm])
            plsc.store_scatter(cur_ref, [s], pos + 1, mask=last)
            return c

        lax.fori_loop(0, nv, pvec, 0)

    pltpu.sync_copy(out_src, srcp_hbm.at[pl.ds(cbase_p, CP)])
    pltpu.sync_copy(out_dst, dstp_hbm.at[pl.ds(cbase_p, CP)])


def _sc_bin(src, dst):
    return pl.kernel(
        _bin_body,
        out_type=[
            jax.ShapeDtypeStruct((EP2,), jnp.int32),
            jax.ShapeDtypeStruct((EP2,), jnp.int32),
            jax.ShapeDtypeStruct((SC_NW, NBIN), jnp.int32),
            jax.ShapeDtypeStruct((SC_NW, NBIN), jnp.int32),
        ],
        mesh=_sc_mesh(),
        compiler_params=pltpu.CompilerParams(use_tc_tiling_on_sc=False,
                                             needs_layout_passes=False),
        scratch_types=[
            pltpu.VMEM((6400,), jnp.int32),
            pltpu.VMEM((6400,), jnp.int32),
            pltpu.VMEM((CP,), jnp.int32),
            pltpu.VMEM((CP,), jnp.int32),
            pltpu.VMEM((48,), jnp.int32),
            pltpu.VMEM((48,), jnp.int32),
            pltpu.VMEM((32,), jnp.int32),
            pltpu.VMEM((32,), jnp.int32),
        ],
    )(src, dst)


def _scatter_body(m_hbm, dstp_hbm, offs_hbm, cnts_hbm, agg_hbm,
                  offs_v, cnts_v, acc, mw, dstw):
    t = _wid()
    nlo = t * BINW
    pltpu.sync_copy(offs_hbm, offs_v.at[pl.ds(0, SC_NW * NBIN)])
    pltpu.sync_copy(cnts_hbm, cnts_v.at[pl.ds(0, SC_NW * NBIN)])
    sent = jnp.float32(SENT) + _iota16() * 0.0

    def init(r, c):
        for h in (0, 1):
            acc[h, r, pl.ds(0, 16)] = sent
            acc[h, r, pl.ds(16, 16)] = sent
        return c

    lax.fori_loop(0, BINW, init, 0)

    def chunk(ci, c):
        off = offs_v[pl.ds(ci * NBIN + t, 16)][0]
        cnt = cnts_v[pl.ds(ci * NBIN + t, 16)][0]
        nw = (cnt + WS - 1) // WS

        def win(w, c2):
            ws = pl.multiple_of(off + w * WS, 8)
            pltpu.sync_copy(dstp_hbm.at[pl.ds(ws, WS)], dstw.at[pl.ds(0, WS)])
            pltpu.sync_copy(m_hbm.at[pl.ds(ws, WS)], mw)
            ne = jnp.minimum(WS, cnt - w * WS)

            def pair(e2, c3):
                for h in (0, 1):
                    idx = 2 * e2 + h

                    @pl.when(idx < ne)
                    def _():
                        rel = dstw[pl.ds(idx, 16)][0] - nlo
                        a0 = acc[h, rel, pl.ds(0, 16)]
                        a1 = acc[h, rel, pl.ds(16, 16)]
                        acc[h, rel, pl.ds(0, 16)] = jnp.maximum(
                            a0, mw[idx, pl.ds(0, 16)])
                        acc[h, rel, pl.ds(16, 16)] = jnp.maximum(
                            a1, mw[idx, pl.ds(16, 16)])
                return c3

            lax.fori_loop(0, (ne + 1) // 2, pair, 0)
            return c2

        lax.fori_loop(0, nw, win, 0)
        return c

    lax.fori_loop(0, SC_NW, chunk, 0)

    def merge(r, c):
        for sl in (pl.ds(0, 16), pl.ds(16, 16)):
            v = jnp.maximum(acc[0, r, sl], acc[1, r, sl])
            acc[0, r, sl] = jnp.where(v > -1.0e37, v, 0.0)
        return c

    lax.fori_loop(0, BINW, merge, 0)
    pltpu.sync_copy(acc.at[0], agg_hbm.at[pl.ds(nlo, BINW)])


def _sc_scatter(m, dst_p, offs, cnts):
    return pl.kernel(
        _scatter_body,
        out_type=jax.ShapeDtypeStruct((NPAD, 32), jnp.float32),
        mesh=_sc_mesh(),
        compiler_params=pltpu.CompilerParams(use_tc_tiling_on_sc=False),
        scratch_types=[
            pltpu.VMEM((SC_NW * NBIN + 16,), jnp.int32),
            pltpu.VMEM((SC_NW * NBIN + 16,), jnp.int32),
            pltpu.VMEM((2, BINW, 32), jnp.float32),
            pltpu.VMEM((WS, 32), jnp.float32),
            pltpu.VMEM((WS + 16,), jnp.int32),
        ],
    )(m, dst_p, offs.reshape(SC_NW * NBIN), cnts.reshape(SC_NW * NBIN))


def _edge_layer(P, Q, src_p, dst_p, offs, cnts, W2, b2):
    g = _sc_gather(P, Q, src_p, dst_p)
    m = _edge_mm(g, W2, b2)
    agg_pad = _sc_scatter(m, dst_p, offs, cnts)
    return agg_pad[:N]


def kernel(x, edge_index, batch, emb_charge, emb_pdgid, emb_frompv, W_cat,
           b_cat, W_cont, b_cont, W_all, b_all, bn_gamma, bn_beta, W1_0, b1_0,
           W2_0, b2_0, W1_1, b1_1, W2_1, b2_1, W_out1, b_out1, W_out2, b_out2):
    W_cat_eff = jnp.concatenate([
        emb_charge @ W_cat[0:8],
        emb_pdgid @ W_cat[8:16],
        emb_frompv @ W_cat[16:24],
    ], axis=0)
    row = lambda b: b.reshape(1, -1)
    src, dst = edge_index[0], edge_index[1]
    src_p, dst_p, offs, cnts = _sc_bin(src, dst)

    h0, s1, s2 = _encode(x, W_cat_eff, row(b_cat), W_cont, row(b_cont),
                         W_all, row(b_all))

    WP0 = W1_0[:32] - W1_0[32:]
    WQ0 = W1_0[32:]
    h, P, Q = _pq_first(h0, s1, s2, row(bn_gamma), row(bn_beta), WP0, WQ0,
                        row(b1_0))
    agg0 = _edge_layer(P, Q, src_p, dst_p, offs, cnts, W2_0, row(b2_0))

    WP1 = W1_1[:32] - W1_1[32:]
    WQ1 = W1_1[32:]
    h, P, Q = _pq_next(h, agg0, WP1, WQ1, row(b1_1))
    agg1 = _edge_layer(P, Q, src_p, dst_p, offs, cnts, W2_1, row(b2_1))

    return _final(h, agg1, W_out1, row(b_out1), W_out2, row(b_out2))


# speedup vs baseline: 2.2628x; 1.0154x over previous
import functools

import jax
import jax.numpy as jnp
from jax import lax
from jax.experimental import pallas as pl
from jax.experimental.pallas import tpu as pltpu
from jax.experimental.pallas import tpu_sc as plsc

N = 50000
E = 800000
H = 32
MESG = 64
PDGS = (1, 2, 11, 13, 22, 130, 211)


def _elu(v):
    return jnp.where(v > 0, v, jnp.exp(jnp.minimum(v, 0.0)) - 1.0)


def _encode_body(x_ref, wcat_ref, bcat_ref, wcont_ref, bcont_ref,
                 wall_ref, ball_ref, h_ref, s1_ref, s2_ref):
    i = pl.program_id(0)
    x = x_ref[...]
    x_cont = x[:, :8]
    pdgv = jnp.abs(x[:, 8:9])
    chv = x[:, 9:10] + 1.0
    fpv = x[:, 10:11]
    oh_ch = jnp.concatenate(
        [(chv == float(v)).astype(jnp.float32) for v in range(3)], axis=1)
    oh_pdg = jnp.concatenate(
        [(pdgv == float(v)).astype(jnp.float32) for v in PDGS], axis=1)
    oh_fp = jnp.concatenate(
        [(fpv == float(v)).astype(jnp.float32) for v in range(4)], axis=1)
    oh = jnp.concatenate([oh_ch, oh_pdg, oh_fp], axis=1)
    emb_cat = _elu(jnp.dot(oh, wcat_ref[...], preferred_element_type=jnp.float32)
                   + bcat_ref[...])
    emb_cont = _elu(jnp.dot(x_cont, wcont_ref[...], preferred_element_type=jnp.float32)
                    + bcont_ref[...])
    hin = jnp.concatenate([emb_cat, emb_cont], axis=1)
    h = _elu(jnp.dot(hin, wall_ref[...], preferred_element_type=jnp.float32)
             + ball_ref[...])
    h_ref[...] = h

    @pl.when(i == 0)
    def _():
        s1_ref[...] = jnp.zeros_like(s1_ref)
        s2_ref[...] = jnp.zeros_like(s2_ref)

    s1_ref[...] += jnp.sum(h, axis=0, keepdims=True)
    s2_ref[...] += jnp.sum(h * h, axis=0, keepdims=True)


def _encode(x, W_cat_eff, b_cat, W_cont, b_cont, W_all, b_all):
    blk = 5000
    grid = N // blk
    full = lambda s: pl.BlockSpec(s, lambda i: (0,) * len(s))
    return pl.pallas_call(
        _encode_body,
        grid=(grid,),
        in_specs=[
            pl.BlockSpec((blk, 11), lambda i: (i, 0)),
            full((14, 16)), full((1, 16)), full((8, 16)), full((1, 16)),
            full((32, 32)), full((1, 32)),
        ],
        out_specs=[
            pl.BlockSpec((blk, 32), lambda i: (i, 0)),
            full((1, 32)), full((1, 32)),
        ],
        out_shape=[
            jax.ShapeDtypeStruct((N, 32), jnp.float32),
            jax.ShapeDtypeStruct((1, 32), jnp.float32),
            jax.ShapeDtypeStruct((1, 32), jnp.float32),
        ],
    )(x, W_cat_eff, b_cat, W_cont, b_cont, W_all, b_all)


def _pq_body(h_ref, s1_ref, s2_ref, g_ref, b_ref, wp_ref, wq_ref, bp_ref,
             h_out_ref, p_ref, q_ref):
    mean = s1_ref[...] / N
    var = s2_ref[...] / N - mean * mean
    inv = g_ref[...] * lax.rsqrt(var + 1e-5)
    h = h_ref[...] * inv + (b_ref[...] - mean * inv)
    h_out_ref[...] = h
    p_ref[...] = jnp.dot(h, wp_ref[...], preferred_element_type=jnp.float32) + bp_ref[...]
    q_ref[...] = jnp.dot(h, wq_ref[...], preferred_element_type=jnp.float32)


def _pq_first(h0, s1, s2, gamma, beta, WP, WQ, b1):
    blk = 5000
    full = lambda s: pl.BlockSpec(s, lambda i: (0,) * len(s))
    return pl.pallas_call(
        _pq_body,
        grid=(N // blk,),
        in_specs=[
            pl.BlockSpec((blk, 32), lambda i: (i, 0)),
            full((1, 32)), full((1, 32)), full((1, 32)), full((1, 32)),
            full((32, 64)), full((32, 64)), full((1, 64)),
        ],
        out_specs=[
            pl.BlockSpec((blk, 32), lambda i: (i, 0)),
            pl.BlockSpec((blk, 64), lambda i: (i, 0)),
            pl.BlockSpec((blk, 64), lambda i: (i, 0)),
        ],
        out_shape=[
            jax.ShapeDtypeStruct((N, 32), jnp.float32),
            jax.ShapeDtypeStruct((N, 64), jnp.float32),
            jax.ShapeDtypeStruct((N, 64), jnp.float32),
        ],
    )(h0, s1, s2, gamma, beta, WP, WQ, b1)


def _pq_next_body(h_ref, agg_ref, wp_ref, wq_ref, bp_ref,
                  h_out_ref, p_ref, q_ref):
    h = h_ref[...] + agg_ref[...]
    h_out_ref[...] = h
    p_ref[...] = jnp.dot(h, wp_ref[...], preferred_element_type=jnp.float32) + bp_ref[...]
    q_ref[...] = jnp.dot(h, wq_ref[...], preferred_element_type=jnp.float32)


def _pq_next(h, agg, WP, WQ, b1):
    blk = 5000
    full = lambda s: pl.BlockSpec(s, lambda i: (0,) * len(s))
    return pl.pallas_call(
        _pq_next_body,
        grid=(N // blk,),
        in_specs=[
            pl.BlockSpec((blk, 32), lambda i: (i, 0)),
            pl.BlockSpec((blk, 32), lambda i: (i, 0)),
            full((32, 64)), full((32, 64)), full((1, 64)),
        ],
        out_specs=[
            pl.BlockSpec((blk, 32), lambda i: (i, 0)),
            pl.BlockSpec((blk, 64), lambda i: (i, 0)),
            pl.BlockSpec((blk, 64), lambda i: (i, 0)),
        ],
        out_shape=[
            jax.ShapeDtypeStruct((N, 32), jnp.float32),
            jax.ShapeDtypeStruct((N, 64), jnp.float32),
            jax.ShapeDtypeStruct((N, 64), jnp.float32),
        ],
    )(h, agg, WP, WQ, b1)


def _mm_body(g_ref, w2_ref, b2_ref, m_ref):
    g = _elu(g_ref[...])
    m_ref[...] = _elu(jnp.dot(g, w2_ref[...],
                              preferred_element_type=jnp.float32) + b2_ref[...])


def _edge_mm(g, W2, b2):
    blk = 6400
    full = lambda s: pl.BlockSpec(s, lambda i: (0,) * len(s))
    return pl.pallas_call(
        _mm_body,
        grid=(EP2 // blk,),
        in_specs=[
            pl.BlockSpec((blk, 64), lambda i: (i, 0)),
            full((64, 32)), full((1, 32)),
        ],
        out_specs=pl.BlockSpec((blk, 32), lambda i: (i, 0)),
        out_shape=jax.ShapeDtypeStruct((EP2, 32), jnp.float32),
    )(g, W2, b2)


def _final_body(h_ref, agg_ref, w1_ref, b1_ref, w2_ref, b2_ref, o_ref):
    h = h_ref[...] + agg_ref[...]
    t = _elu(jnp.dot(h, w1_ref[...], preferred_element_type=jnp.float32) + b1_ref[...])
    o = jnp.dot(t, w2_ref[...], preferred_element_type=jnp.float32) + b2_ref[...]
    o_ref[...] = jax.nn.sigmoid(o)


def _final(h, agg, W_out1, b_out1, W_out2, b_out2):
    blk = 5000
    full = lambda s: pl.BlockSpec(s, lambda i: (0,) * len(s))
    out2 = pl.pallas_call(
        _final_body,
        grid=(N // blk,),
        in_specs=[
            pl.BlockSpec((blk, 32), lambda i: (i, 0)),
            pl.BlockSpec((blk, 32), lambda i: (i, 0)),
            full((32, 16)), full((1, 16)), full((16, 1)), full((1, 1)),
        ],
        out_specs=pl.BlockSpec((blk, 1), lambda i: (i, 0)),
        out_shape=jax.ShapeDtypeStruct((N, 1), jnp.float32),
    )(h, agg, W_out1, b_out1, W_out2, b_out2)
    return out2[:, 0]


SC_NC, SC_NS = 2, 16
SC_NW = SC_NC * SC_NS
C = E // SC_NW
CP = 25600
EPAD = SC_NW * CP
EP2 = EPAD + 6400
EPW = CP
GW = 200
NWIN = EPW // GW
NBIN = 32
BINW = 1563
NPAD = NBIN * BINW
WS = 512
SENT = -3.0e38


def _sc_mesh():
    return plsc.VectorSubcoreMesh(core_axis_name="c", subcore_axis_name="s",
                                  num_cores=SC_NC, num_subcores=SC_NS)


def _wid():
    return lax.axis_index("s") * SC_NC + lax.axis_index("c")


def _iota16():
    return lax.iota(jnp.int32, 16)


def _bin_of(n):
    return lax.shift_right_logical(n * 42936, 26)


def _gather_body(p_hbm, q_hbm, src_hbm, dst_hbm, out_hbm,
                 idx_s, idx_d, rows_p, rows_q, g_buf,
                 sem_i0, sem_i1, sem_p0, sem_p1, sem_q0, sem_q1,
                 sem_o0, sem_o1):
    wid = lax.axis_index("s") * SC_NC + lax.axis_index("c")
    base = wid * EPW
    sem_i = (sem_i0, sem_i1)
    sem_p = (sem_p0, sem_p1)
    sem_q = (sem_q0, sem_q1)
    sem_o = (sem_o0, sem_o1)

    def start_idx(w, s):
        b = base + w * GW
        pltpu.async_copy(src_hbm.at[pl.ds(b, GW)], idx_s.at[s], sem_i[s])
        pltpu.async_copy(dst_hbm.at[pl.ds(b, GW)], idx_d.at[s], sem_i[s])

    def wait_idx(s):
        pltpu.make_async_copy(src_hbm.at[pl.ds(base, GW)], idx_s.at[s],
                              sem_i[s]).wait()
        pltpu.make_async_copy(dst_hbm.at[pl.ds(base, GW)], idx_d.at[s],
                              sem_i[s]).wait()

    def start_rows(s):
        pltpu.async_copy(q_hbm.at[idx_s.at[s]], rows_q.at[s], sem_q[s])
        pltpu.async_copy(p_hbm.at[idx_d.at[s]], rows_p.at[s], sem_p[s])

    def wait_rows(s):
        pltpu.make_async_copy(q_hbm.at[idx_s.at[s]], rows_q.at[s],
                              sem_q[s]).wait()
        pltpu.make_async_copy(p_hbm.at[idx_d.at[s]], rows_p.at[s],
                              sem_p[s]).wait()

    def start_out(w, s):
        b = base + w * GW
        pltpu.async_copy(g_buf.at[s], out_hbm.at[pl.ds(b, GW)], sem_o[s])

    def wait_out(s):
        pltpu.make_async_copy(g_buf.at[s], out_hbm.at[pl.ds(base, GW)],
                              sem_o[s]).wait()

    def compute(s):
        def edge(e, c):
            for j in range(4):
                sl = pl.ds(j * 16, 16)
                g_buf[s, e, sl] = rows_p[s, e, sl] + rows_q[s, e, sl]
            return c

        lax.fori_loop(0, GW, edge, 0, unroll=4)

    start_idx(0, 0)
    wait_idx(0)
    start_rows(0)
    start_idx(1, 1)

    def half(w, s, o):
        @pl.when(w < NWIN)
        def _():
            @pl.when(w + 1 < NWIN)
            def _():
                wait_idx(o)
                start_rows(o)

            wait_rows(s)

            @pl.when(w + 2 < NWIN)
            def _():
                start_idx(w + 2, s)

            @pl.when(w >= 2)
            def _():
                wait_out(s)

            compute(s)
            start_out(w, s)

    def body(i, c):
        half(2 * i, 0, 1)
        half(2 * i + 1, 1, 0)
        return c

    lax.fori_loop(0, (NWIN + 1) // 2, body, 0)
    wait_out(0)
    wait_out(1)


def _sc_gather(P, Q, src, dst):
    mesh = plsc.VectorSubcoreMesh(core_axis_name="c", subcore_axis_name="s",
                                  num_cores=SC_NC, num_subcores=SC_NS)
    return pl.kernel(
        _gather_body,
        out_type=jax.ShapeDtypeStruct((EP2, 64), jnp.float32),
        mesh=mesh,
        compiler_params=pltpu.CompilerParams(use_tc_tiling_on_sc=False),
        scratch_types=[
            pltpu.VMEM((2, GW), jnp.int32),
            pltpu.VMEM((2, GW), jnp.int32),
            pltpu.VMEM((2, GW, 64), jnp.float32),
            pltpu.VMEM((2, GW, 64), jnp.float32),
            pltpu.VMEM((2, GW, 64), jnp.float32),
        ] + [pltpu.SemaphoreType.DMA] * 8,
    )(P, Q, src, dst)


_BIN_WINS = ((0, 6400), (6400, 6400), (12800, 6400), (19200, 5800))


def _runs(s):
    i16 = _iota16()
    prev = s[jnp.maximum(i16 - 1, 0)]
    first = (i16 == 0) | (s != prev)
    nxt = s[jnp.minimum(i16 + 1, 15)]
    last = (i16 == 15) | (s != nxt)
    pstart = plsc.cummax(jnp.where(first, i16, 0))
    rank = i16 - pstart
    return first, last, rank


def _bin_body(src_hbm, dst_hbm, srcp_hbm, dstp_hbm, offs_hbm, cnts_hbm,
              dstw, srcw, out_src, out_dst, cnt_ref, cur_ref, obuf, cbuf):
    cw = _wid()
    i16 = _iota16()
    zero16 = i16 * 0
    cbase_raw = cw * C
    cbase_p = cw * CP

    cnt_ref[pl.ds(0, 16)] = zero16
    cnt_ref[pl.ds(16, 16)] = zero16
    cnt_ref[pl.ds(32, 16)] = zero16

    def pre(v, c):
        out_src[pl.ds(v * 16, 16)] = zero16
        out_dst[pl.ds(v * 16, 16)] = zero16
        return c

    lax.fori_loop(0, CP // 16, pre, 0)

    for wb, wlen in _BIN_WINS:
        pltpu.sync_copy(dst_hbm.at[pl.ds(cbase_raw + wb, wlen)],
                        dstw.at[pl.ds(0, wlen)])
        nv = (wlen + 15) // 16

        def cvec(v, c, wb=wb):
            d = dstw[pl.ds(v * 16, 16)]
            b = jnp.where(i16 < (C - wb - v * 16), _bin_of(d), NBIN)
            s, _ = plsc.sort_key_val(b, i16)
            _, last, rank = _runs(s)
            cur = plsc.load_gather(cnt_ref, [s])
            plsc.store_scatter(cnt_ref, [s], cur + rank + 1, mask=last)
            return c

        lax.fori_loop(0, nv, cvec, 0)

    cnt0 = cnt_ref[pl.ds(0, 16)]
    cnt1 = cnt_ref[pl.ds(16, 16)]
    r0 = jnp.bitwise_and(cnt0 + 7, -8)
    r1 = jnp.bitwise_and(cnt1 + 7, -8)
    c0 = plsc.cumsum(r0)
    c1 = plsc.cumsum(r1)
    tot0 = c0[zero16 + 15]
    excl0 = c0 - r0
    excl1 = c1 - r1 + tot0
    end_all = c1[zero16 + 15] + tot0
    cur_ref[pl.ds(0, 16)] = excl0
    cur_ref[pl.ds(16, 16)] = excl1
    cur_ref[pl.ds(32, 16)] = end_all + zero16
    obuf[pl.ds(0, 16)] = excl0 + cbase_p
    obuf[pl.ds(16, 16)] = excl1 + cbase_p
    cbuf[pl.ds(0, 16)] = cnt0
    cbuf[pl.ds(16, 16)] = cnt1
    pltpu.sync_copy(obuf, offs_hbm.at[cw])
    pltpu.sync_copy(cbuf, cnts_hbm.at[cw])

    for wb, wlen in _BIN_WINS:
        pltpu.sync_copy(dst_hbm.at[pl.ds(cbase_raw + wb, wlen)],
                        dstw.at[pl.ds(0, wlen)])
        pltpu.sync_copy(src_hbm.at[pl.ds(cbase_raw + wb, wlen)],
                        srcw.at[pl.ds(0, wlen)])
        nv = (wlen + 15) // 16

        def pvec(v, c, wb=wb):
            d = dstw[pl.ds(v * 16, 16)]
            sv = srcw[pl.ds(v * 16, 16)]
            valid = i16 < (C - wb - v * 16)
            b = jnp.where(valid, _bin_of(d), NBIN)
            s, perm = plsc.sort_key_val(b, i16)
            _, last, rank = _runs(s)
            cur = plsc.load_gather(cur_ref, [s])
            pos = cur + rank
            plsc.store_scatter(out_dst, [pos], jnp.where(valid, d, 0)[perm])
            plsc.store_scatter(out_src, [pos], jnp.where(valid, sv, 0)[perm])
            plsc.store_scatter(cur_ref, [s], pos + 1, mask=last)
            return c

        lax.fori_loop(0, nv, pvec, 0)

    pltpu.sync_copy(out_src, srcp_hbm.at[pl.ds(cbase_p, CP)])
    pltpu.sync_copy(out_dst, dstp_hbm.at[pl.ds(cbase_p, CP)])


def _sc_bin(src, dst):
    return pl.kernel(
        _bin_body,
        out_type=[
            jax.ShapeDtypeStruct((EP2,), jnp.int32),
            jax.ShapeDtypeStruct((EP2,), jnp.int32),
            jax.ShapeDtypeStruct((SC_NW, NBIN), jnp.int32),
            jax.ShapeDtypeStruct((SC_NW, NBIN), jnp.int32),
        ],
        mesh=_sc_mesh(),
        compiler_params=pltpu.CompilerParams(use_tc_tiling_on_sc=False,
                                             needs_layout_passes=False),
        scratch_types=[
            pltpu.VMEM((6400,), jnp.int32),
            pltpu.VMEM((6400,), jnp.int32),
            pltpu.VMEM((CP,), jnp.int32),
            pltpu.VMEM((CP,), jnp.int32),
            pltpu.VMEM((48,), jnp.int32),
            pltpu.VMEM((48,), jnp.int32),
            pltpu.VMEM((32,), jnp.int32),
            pltpu.VMEM((32,), jnp.int32),
        ],
    )(src, dst)


def _scatter_body(m_hbm, dstp_hbm, offs_hbm, cnts_hbm, agg_hbm,
                  offs_v, cnts_v, acc, mw, dstw):
    t = _wid()
    nlo = t * BINW
    pltpu.sync_copy(offs_hbm, offs_v.at[pl.ds(0, SC_NW * NBIN)])
    pltpu.sync_copy(cnts_hbm, cnts_v.at[pl.ds(0, SC_NW * NBIN)])
    sent = jnp.float32(SENT) + _iota16() * 0.0

    def init(r, c):
        for h in (0, 1):
            acc[h, r, pl.ds(0, 16)] = sent
            acc[h, r, pl.ds(16, 16)] = sent
        return c

    lax.fori_loop(0, BINW, init, 0)

    def update(idx, h):
        rel = dstw[pl.ds(idx, 16)][0] - nlo
        a0 = acc[h, rel, pl.ds(0, 16)]
        a1 = acc[h, rel, pl.ds(16, 16)]
        acc[h, rel, pl.ds(0, 16)] = jnp.maximum(a0, mw[idx, pl.ds(0, 16)])
        acc[h, rel, pl.ds(16, 16)] = jnp.maximum(a1, mw[idx, pl.ds(16, 16)])

    def chunk(ci, c):
        off = offs_v[pl.ds(ci * NBIN + t, 16)][0]
        cnt = cnts_v[pl.ds(ci * NBIN + t, 16)][0]
        nfull = cnt // WS

        def load_win(w):
            ws = pl.multiple_of(off + w * WS, 8)
            pltpu.sync_copy(dstp_hbm.at[pl.ds(ws, WS)], dstw.at[pl.ds(0, WS)])
            pltpu.sync_copy(m_hbm.at[pl.ds(ws, WS)], mw)

        def win_full(w, c2):
            load_win(w)

            def pair(e2, c3):
                update(2 * e2, 0)
                update(2 * e2 + 1, 1)
                return c3

            lax.fori_loop(0, WS // 2, pair, 0)
            return c2

        lax.fori_loop(0, nfull, win_full, 0)
        ne = cnt - nfull * WS

        @pl.when(ne > 0)
        def _():
            load_win(nfull)

            def pair(e2, c3):
                for h in (0, 1):
                    idx = 2 * e2 + h

                    @pl.when(idx < ne)
                    def _():
                        update(idx, h)
                return c3

            lax.fori_loop(0, (ne + 1) // 2, pair, 0)

        return c

    lax.fori_loop(0, SC_NW, chunk, 0)

    def merge(r, c):
        for sl in (pl.ds(0, 16), pl.ds(16, 16)):
            v = jnp.maximum(acc[0, r, sl], acc[1, r, sl])
            acc[0, r, sl] = jnp.where(v > -1.0e37, v, 0.0)
        return c

    lax.fori_loop(0, BINW, merge, 0)
    pltpu.sync_copy(acc.at[0], agg_hbm.at[pl.ds(nlo, BINW)])


def _sc_scatter(m, dst_p, offs, cnts):
    return pl.kernel(
        _scatter_body,
        out_type=jax.ShapeDtypeStruct((NPAD, 32), jnp.float32),
        mesh=_sc_mesh(),
        compiler_params=pltpu.CompilerParams(use_tc_tiling_on_sc=False),
        scratch_types=[
            pltpu.VMEM((SC_NW * NBIN + 16,), jnp.int32),
            pltpu.VMEM((SC_NW * NBIN + 16,), jnp.int32),
            pltpu.VMEM((2, BINW, 32), jnp.float32),
            pltpu.VMEM((WS, 32), jnp.float32),
            pltpu.VMEM((WS + 16,), jnp.int32),
        ],
    )(m, dst_p, offs.reshape(SC_NW * NBIN), cnts.reshape(SC_NW * NBIN))


def _edge_layer(P, Q, src_p, dst_p, offs, cnts, W2, b2):
    g = _sc_gather(P, Q, src_p, dst_p)
    m = _edge_mm(g, W2, b2)
    agg_pad = _sc_scatter(m, dst_p, offs, cnts)
    return agg_pad[:N]


def kernel(x, edge_index, batch, emb_charge, emb_pdgid, emb_frompv, W_cat,
           b_cat, W_cont, b_cont, W_all, b_all, bn_gamma, bn_beta, W1_0, b1_0,
           W2_0, b2_0, W1_1, b1_1, W2_1, b2_1, W_out1, b_out1, W_out2, b_out2):
    W_cat_eff = jnp.concatenate([
        emb_charge @ W_cat[0:8],
        emb_pdgid @ W_cat[8:16],
        emb_frompv @ W_cat[16:24],
    ], axis=0)
    row = lambda b: b.reshape(1, -1)
    src, dst = edge_index[0], edge_index[1]
    src_p, dst_p, offs, cnts = _sc_bin(src, dst)

    h0, s1, s2 = _encode(x, W_cat_eff, row(b_cat), W_cont, row(b_cont),
                         W_all, row(b_all))

    WP0 = W1_0[:32] - W1_0[32:]
    WQ0 = W1_0[32:]
    h, P, Q = _pq_first(h0, s1, s2, row(bn_gamma), row(bn_beta), WP0, WQ0,
                        row(b1_0))
    agg0 = _edge_layer(P, Q, src_p, dst_p, offs, cnts, W2_0, row(b2_0))

    WP1 = W1_1[:32] - W1_1[32:]
    WQ1 = W1_1[32:]
    h, P, Q = _pq_next(h, agg0, WP1, WQ1, row(b1_1))
    agg1 = _edge_layer(P, Q, src_p, dst_p, offs, cnts, W2_1, row(b2_1))

    return _final(h, agg1, W_out1, row(b_out1), W_out2, row(b_out2))


# speedup vs baseline: 2.4650x; 1.0893x over previous
import functools

import jax
import jax.numpy as jnp
from jax import lax
from jax.experimental import pallas as pl
from jax.experimental.pallas import tpu as pltpu
from jax.experimental.pallas import tpu_sc as plsc

N = 50000
E = 800000
H = 32
MESG = 64
PDGS = (1, 2, 11, 13, 22, 130, 211)


def _elu(v):
    return jnp.where(v > 0, v, jnp.exp(jnp.minimum(v, 0.0)) - 1.0)


def _encode_body(x_ref, wcat_ref, bcat_ref, wcont_ref, bcont_ref,
                 wall_ref, ball_ref, h_ref, s1_ref, s2_ref):
    i = pl.program_id(0)
    x = x_ref[...]
    x_cont = x[:, :8]
    pdgv = jnp.abs(x[:, 8:9])
    chv = x[:, 9:10] + 1.0
    fpv = x[:, 10:11]
    oh_ch = jnp.concatenate(
        [(chv == float(v)).astype(jnp.float32) for v in range(3)], axis=1)
    oh_pdg = jnp.concatenate(
        [(pdgv == float(v)).astype(jnp.float32) for v in PDGS], axis=1)
    oh_fp = jnp.concatenate(
        [(fpv == float(v)).astype(jnp.float32) for v in range(4)], axis=1)
    oh = jnp.concatenate([oh_ch, oh_pdg, oh_fp], axis=1)
    emb_cat = _elu(jnp.dot(oh, wcat_ref[...], preferred_element_type=jnp.float32)
                   + bcat_ref[...])
    emb_cont = _elu(jnp.dot(x_cont, wcont_ref[...], preferred_element_type=jnp.float32)
                    + bcont_ref[...])
    hin = jnp.concatenate([emb_cat, emb_cont], axis=1)
    h = _elu(jnp.dot(hin, wall_ref[...], preferred_element_type=jnp.float32)
             + ball_ref[...])
    h_ref[...] = h

    @pl.when(i == 0)
    def _():
        s1_ref[...] = jnp.zeros_like(s1_ref)
        s2_ref[...] = jnp.zeros_like(s2_ref)

    s1_ref[...] += jnp.sum(h, axis=0, keepdims=True)
    s2_ref[...] += jnp.sum(h * h, axis=0, keepdims=True)


def _encode(x, W_cat_eff, b_cat, W_cont, b_cont, W_all, b_all):
    blk = 5000
    grid = N // blk
    full = lambda s: pl.BlockSpec(s, lambda i: (0,) * len(s))
    return pl.pallas_call(
        _encode_body,
        grid=(grid,),
        in_specs=[
            pl.BlockSpec((blk, 11), lambda i: (i, 0)),
            full((14, 16)), full((1, 16)), full((8, 16)), full((1, 16)),
            full((32, 32)), full((1, 32)),
        ],
        out_specs=[
            pl.BlockSpec((blk, 32), lambda i: (i, 0)),
            full((1, 32)), full((1, 32)),
        ],
        out_shape=[
            jax.ShapeDtypeStruct((N, 32), jnp.float32),
            jax.ShapeDtypeStruct((1, 32), jnp.float32),
            jax.ShapeDtypeStruct((1, 32), jnp.float32),
        ],
    )(x, W_cat_eff, b_cat, W_cont, b_cont, W_all, b_all)


def _pq_body(h_ref, s1_ref, s2_ref, g_ref, b_ref, wp_ref, wq_ref, bp_ref,
             h_out_ref, p_ref, q_ref):
    mean = s1_ref[...] / N
    var = s2_ref[...] / N - mean * mean
    inv = g_ref[...] * lax.rsqrt(var + 1e-5)
    h = h_ref[...] * inv + (b_ref[...] - mean * inv)
    h_out_ref[...] = h
    p_ref[...] = jnp.dot(h, wp_ref[...], preferred_element_type=jnp.float32) + bp_ref[...]
    q_ref[...] = jnp.dot(h, wq_ref[...], preferred_element_type=jnp.float32)


def _pq_first(h0, s1, s2, gamma, beta, WP, WQ, b1):
    blk = 5000
    full = lambda s: pl.BlockSpec(s, lambda i: (0,) * len(s))
    return pl.pallas_call(
        _pq_body,
        grid=(N // blk,),
        in_specs=[
            pl.BlockSpec((blk, 32), lambda i: (i, 0)),
            full((1, 32)), full((1, 32)), full((1, 32)), full((1, 32)),
            full((32, 64)), full((32, 64)), full((1, 64)),
        ],
        out_specs=[
            pl.BlockSpec((blk, 32), lambda i: (i, 0)),
            pl.BlockSpec((blk, 64), lambda i: (i, 0)),
            pl.BlockSpec((blk, 64), lambda i: (i, 0)),
        ],
        out_shape=[
            jax.ShapeDtypeStruct((N, 32), jnp.float32),
            jax.ShapeDtypeStruct((N, 64), jnp.float32),
            jax.ShapeDtypeStruct((N, 64), jnp.float32),
        ],
    )(h0, s1, s2, gamma, beta, WP, WQ, b1)


def _pq_next_body(h_ref, agg_ref, wp_ref, wq_ref, bp_ref,
                  h_out_ref, p_ref, q_ref):
    h = h_ref[...] + agg_ref[...]
    h_out_ref[...] = h
    p_ref[...] = jnp.dot(h, wp_ref[...], preferred_element_type=jnp.float32) + bp_ref[...]
    q_ref[...] = jnp.dot(h, wq_ref[...], preferred_element_type=jnp.float32)


def _pq_next(h, agg, WP, WQ, b1):
    blk = 5000
    full = lambda s: pl.BlockSpec(s, lambda i: (0,) * len(s))
    return pl.pallas_call(
        _pq_next_body,
        grid=(N // blk,),
        in_specs=[
            pl.BlockSpec((blk, 32), lambda i: (i, 0)),
            pl.BlockSpec((blk, 32), lambda i: (i, 0)),
            full((32, 64)), full((32, 64)), full((1, 64)),
        ],
        out_specs=[
            pl.BlockSpec((blk, 32), lambda i: (i, 0)),
            pl.BlockSpec((blk, 64), lambda i: (i, 0)),
            pl.BlockSpec((blk, 64), lambda i: (i, 0)),
        ],
        out_shape=[
            jax.ShapeDtypeStruct((N, 32), jnp.float32),
            jax.ShapeDtypeStruct((N, 64), jnp.float32),
            jax.ShapeDtypeStruct((N, 64), jnp.float32),
        ],
    )(h, agg, WP, WQ, b1)


def _mm_body(g_ref, w2_ref, b2_ref, m_ref):
    g = _elu(g_ref[...])
    m_ref[...] = _elu(jnp.dot(g, w2_ref[...],
                              preferred_element_type=jnp.float32) + b2_ref[...])


def _edge_mm(g, W2, b2):
    blk = 6400
    full = lambda s: pl.BlockSpec(s, lambda i: (0,) * len(s))
    return pl.pallas_call(
        _mm_body,
        grid=(EP2 // blk,),
        in_specs=[
            pl.BlockSpec((blk, 64), lambda i: (i, 0)),
            full((64, 32)), full((1, 32)),
        ],
        out_specs=pl.BlockSpec((blk, 32), lambda i: (i, 0)),
        out_shape=jax.ShapeDtypeStruct((EP2, 32), jnp.float32),
    )(g, W2, b2)


def _final_body(h_ref, agg_ref, w1_ref, b1_ref, w2_ref, b2_ref, o_ref):
    h = h_ref[...] + agg_ref[...]
    t = _elu(jnp.dot(h, w1_ref[...], preferred_element_type=jnp.float32) + b1_ref[...])
    o = jnp.dot(t, w2_ref[...], preferred_element_type=jnp.float32) + b2_ref[...]
    o_ref[...] = jax.nn.sigmoid(o)


def _final(h, agg, W_out1, b_out1, W_out2, b_out2):
    blk = 5000
    full = lambda s: pl.BlockSpec(s, lambda i: (0,) * len(s))
    out2 = pl.pallas_call(
        _final_body,
        grid=(N // blk,),
        in_specs=[
            pl.BlockSpec((blk, 32), lambda i: (i, 0)),
            pl.BlockSpec((blk, 32), lambda i: (i, 0)),
            full((32, 16)), full((1, 16)), full((16, 1)), full((1, 1)),
        ],
        out_specs=pl.BlockSpec((blk, 1), lambda i: (i, 0)),
        out_shape=jax.ShapeDtypeStruct((N, 1), jnp.float32),
    )(h, agg, W_out1, b_out1, W_out2, b_out2)
    return out2[:, 0]


SC_NC, SC_NS = 2, 16
SC_NW = SC_NC * SC_NS
C = E // SC_NW
CP = 25600
EPAD = SC_NW * CP
EP2 = EPAD + 6400
EPW = CP
GW = 256
NWIN = EPW // GW
NBIN = 32
BINW = 1563
NPAD = NBIN * BINW
WS = 512
SENT = -3.0e38


def _sc_mesh():
    return plsc.VectorSubcoreMesh(core_axis_name="c", subcore_axis_name="s",
                                  num_cores=SC_NC, num_subcores=SC_NS)


def _wid():
    return lax.axis_index("s") * SC_NC + lax.axis_index("c")


def _iota16():
    return lax.iota(jnp.int32, 16)


def _bin_of(n):
    return lax.shift_right_logical(n * 42936, 26)


def _gather_body(p_hbm, q_hbm, src_hbm, dst_hbm, out_hbm,
                 idx_s, idx_d, rows_p, rows_q, g_buf,
                 sem_i0, sem_i1, sem_p0, sem_p1, sem_q0, sem_q1,
                 sem_o0, sem_o1):
    wid = lax.axis_index("s") * SC_NC + lax.axis_index("c")
    base = wid * EPW
    sem_i = (sem_i0, sem_i1)
    sem_p = (sem_p0, sem_p1)
    sem_q = (sem_q0, sem_q1)
    sem_o = (sem_o0, sem_o1)

    def start_idx(w, s):
        b = base + w * GW
        pltpu.async_copy(src_hbm.at[pl.ds(b, GW)], idx_s.at[s], sem_i[s])
        pltpu.async_copy(dst_hbm.at[pl.ds(b, GW)], idx_d.at[s], sem_i[s])

    def wait_idx(s):
        pltpu.make_async_copy(src_hbm.at[pl.ds(base, GW)], idx_s.at[s],
                              sem_i[s]).wait()
        pltpu.make_async_copy(dst_hbm.at[pl.ds(base, GW)], idx_d.at[s],
                              sem_i[s]).wait()

    def start_rows(s):
        pltpu.async_copy(q_hbm.at[idx_s.at[s]], rows_q.at[s], sem_q[s])
        pltpu.async_copy(p_hbm.at[idx_d.at[s]], rows_p.at[s], sem_p[s])

    def wait_rows(s):
        pltpu.make_async_copy(q_hbm.at[idx_s.at[s]], rows_q.at[s],
                              sem_q[s]).wait()
        pltpu.make_async_copy(p_hbm.at[idx_d.at[s]], rows_p.at[s],
                              sem_p[s]).wait()

    def start_out(w, s):
        b = base + w * GW
        pltpu.async_copy(g_buf.at[s], out_hbm.at[pl.ds(b, GW)], sem_o[s])

    def wait_out(s):
        pltpu.make_async_copy(g_buf.at[s], out_hbm.at[pl.ds(base, GW)],
                              sem_o[s]).wait()

    def compute(s):
        @plsc.parallel_loop(0, GW, unroll=4)
        def _(e):
            for j in range(4):
                sl = pl.ds(j * 16, 16)
                g_buf[s, e, sl] = rows_p[s, e, sl] + rows_q[s, e, sl]

    start_idx(0, 0)
    wait_idx(0)
    start_rows(0)
    start_idx(1, 1)

    def half(w, s, o):
        @pl.when(w < NWIN)
        def _():
            @pl.when(w + 1 < NWIN)
            def _():
                wait_idx(o)
                start_rows(o)

            wait_rows(s)

            @pl.when(w + 2 < NWIN)
            def _():
                start_idx(w + 2, s)

            @pl.when(w >= 2)
            def _():
                wait_out(s)

            compute(s)
            start_out(w, s)

    def body(i, c):
        half(2 * i, 0, 1)
        half(2 * i + 1, 1, 0)
        return c

    lax.fori_loop(0, (NWIN + 1) // 2, body, 0)
    wait_out(0)
    wait_out(1)


def _sc_gather(P, Q, src, dst):
    mesh = plsc.VectorSubcoreMesh(core_axis_name="c", subcore_axis_name="s",
                                  num_cores=SC_NC, num_subcores=SC_NS)
    return pl.kernel(
        _gather_body,
        out_type=jax.ShapeDtypeStruct((EP2, 64), jnp.float32),
        mesh=mesh,
        compiler_params=pltpu.CompilerParams(use_tc_tiling_on_sc=False),
        scratch_types=[
            pltpu.VMEM((2, GW), jnp.int32),
            pltpu.VMEM((2, GW), jnp.int32),
            pltpu.VMEM((2, GW, 64), jnp.float32),
            pltpu.VMEM((2, GW, 64), jnp.float32),
            pltpu.VMEM((2, GW, 64), jnp.float32),
        ] + [pltpu.SemaphoreType.DMA] * 8,
    )(P, Q, src, dst)


_BIN_WINS = ((0, 6400), (6400, 6400), (12800, 6400), (19200, 5800))


def _runs(s):
    i16 = _iota16()
    prev = s[jnp.maximum(i16 - 1, 0)]
    first = (i16 == 0) | (s != prev)
    nxt = s[jnp.minimum(i16 + 1, 15)]
    last = (i16 == 15) | (s != nxt)
    pstart = plsc.cummax(jnp.where(first, i16, 0))
    rank = i16 - pstart
    return first, last, rank


def _bin_body(src_hbm, dst_hbm, srcp_hbm, dstp_hbm, offs_hbm, cnts_hbm,
              dstw, srcw, out_src, out_dst, cnt_ref, cur_ref, obuf, cbuf):
    cw = _wid()
    i16 = _iota16()
    zero16 = i16 * 0
    cbase_raw = cw * C
    cbase_p = cw * CP

    cnt_ref[pl.ds(0, 16)] = zero16
    cnt_ref[pl.ds(16, 16)] = zero16
    cnt_ref[pl.ds(32, 16)] = zero16

    def pre(v, c):
        out_src[pl.ds(v * 16, 16)] = zero16
        out_dst[pl.ds(v * 16, 16)] = zero16
        return c

    lax.fori_loop(0, CP // 16, pre, 0)

    for wb, wlen in _BIN_WINS:
        pltpu.sync_copy(dst_hbm.at[pl.ds(cbase_raw + wb, wlen)],
                        dstw.at[pl.ds(0, wlen)])
        nv = (wlen + 15) // 16

        def cvec(v, c, wb=wb):
            d = dstw[pl.ds(v * 16, 16)]
            b = jnp.where(i16 < (C - wb - v * 16), _bin_of(d), NBIN)
            s, _ = plsc.sort_key_val(b, i16)
            _, last, rank = _runs(s)
            cur = plsc.load_gather(cnt_ref, [s])
            plsc.store_scatter(cnt_ref, [s], cur + rank + 1, mask=last)
            return c

        lax.fori_loop(0, nv, cvec, 0)

    cnt0 = cnt_ref[pl.ds(0, 16)]
    cnt1 = cnt_ref[pl.ds(16, 16)]
    r0 = jnp.bitwise_and(cnt0 + 7, -8)
    r1 = jnp.bitwise_and(cnt1 + 7, -8)
    c0 = plsc.cumsum(r0)
    c1 = plsc.cumsum(r1)
    tot0 = c0[zero16 + 15]
    excl0 = c0 - r0
    excl1 = c1 - r1 + tot0
    end_all = c1[zero16 + 15] + tot0
    cur_ref[pl.ds(0, 16)] = excl0
    cur_ref[pl.ds(16, 16)] = excl1
    cur_ref[pl.ds(32, 16)] = end_all + zero16
    obuf[pl.ds(0, 16)] = excl0 + cbase_p
    obuf[pl.ds(16, 16)] = excl1 + cbase_p
    cbuf[pl.ds(0, 16)] = cnt0
    cbuf[pl.ds(16, 16)] = cnt1
    pltpu.sync_copy(obuf, offs_hbm.at[cw])
    pltpu.sync_copy(cbuf, cnts_hbm.at[cw])

    for wb, wlen in _BIN_WINS:
        pltpu.sync_copy(dst_hbm.at[pl.ds(cbase_raw + wb, wlen)],
                        dstw.at[pl.ds(0, wlen)])
        pltpu.sync_copy(src_hbm.at[pl.ds(cbase_raw + wb, wlen)],
                        srcw.at[pl.ds(0, wlen)])
        nv = (wlen + 15) // 16

        def pvec(v, c, wb=wb):
            d = dstw[pl.ds(v * 16, 16)]
            sv = srcw[pl.ds(v * 16, 16)]
            valid = i16 < (C - wb - v * 16)
            b = jnp.where(valid, _bin_of(d), NBIN)
            s, perm = plsc.sort_key_val(b, i16)
            _, last, rank = _runs(s)
            cur = plsc.load_gather(cur_ref, [s])
            pos = cur + rank
            plsc.store_scatter(out_dst, [pos], jnp.where(valid, d, 0)[perm])
            plsc.store_scatter(out_src, [pos], jnp.where(valid, sv, 0)[perm])
            plsc.store_scatter(cur_ref, [s], pos + 1, mask=last)
            return c

        lax.fori_loop(0, nv, pvec, 0)

    pltpu.sync_copy(out_src, srcp_hbm.at[pl.ds(cbase_p, CP)])
    pltpu.sync_copy(out_dst, dstp_hbm.at[pl.ds(cbase_p, CP)])


def _sc_bin(src, dst):
    return pl.kernel(
        _bin_body,
        out_type=[
            jax.ShapeDtypeStruct((EP2,), jnp.int32),
            jax.ShapeDtypeStruct((EP2,), jnp.int32),
            jax.ShapeDtypeStruct((SC_NW, NBIN), jnp.int32),
            jax.ShapeDtypeStruct((SC_NW, NBIN), jnp.int32),
        ],
        mesh=_sc_mesh(),
        compiler_params=pltpu.CompilerParams(use_tc_tiling_on_sc=False,
                                             needs_layout_passes=False),
        scratch_types=[
            pltpu.VMEM((6400,), jnp.int32),
            pltpu.VMEM((6400,), jnp.int32),
            pltpu.VMEM((CP,), jnp.int32),
            pltpu.VMEM((CP,), jnp.int32),
            pltpu.VMEM((48,), jnp.int32),
            pltpu.VMEM((48,), jnp.int32),
            pltpu.VMEM((32,), jnp.int32),
            pltpu.VMEM((32,), jnp.int32),
        ],
    )(src, dst)


def _scatter_body(m_hbm, dstp_hbm, offs_hbm, cnts_hbm, agg_hbm,
                  offs_v, cnts_v, acc, mw, dstw):
    t = _wid()
    nlo = t * BINW
    pltpu.sync_copy(offs_hbm, offs_v.at[pl.ds(0, SC_NW * NBIN)])
    pltpu.sync_copy(cnts_hbm, cnts_v.at[pl.ds(0, SC_NW * NBIN)])
    sent = jnp.float32(SENT) + _iota16() * 0.0

    def init(r, c):
        for h in (0, 1):
            acc[h, r, pl.ds(0, 16)] = sent
            acc[h, r, pl.ds(16, 16)] = sent
        return c

    lax.fori_loop(0, BINW, init, 0)

    def update(idx, h):
        rel = dstw[pl.ds(idx, 16)][0] - nlo
        a0 = acc[h, rel, pl.ds(0, 16)]
        a1 = acc[h, rel, pl.ds(16, 16)]
        acc[h, rel, pl.ds(0, 16)] = jnp.maximum(a0, mw[idx, pl.ds(0, 16)])
        acc[h, rel, pl.ds(16, 16)] = jnp.maximum(a1, mw[idx, pl.ds(16, 16)])

    def chunk(ci, c):
        off = offs_v[pl.ds(ci * NBIN + t, 16)][0]
        cnt = cnts_v[pl.ds(ci * NBIN + t, 16)][0]
        nfull = cnt // WS

        def load_win(w):
            ws = pl.multiple_of(off + w * WS, 8)
            pltpu.sync_copy(dstp_hbm.at[pl.ds(ws, WS)], dstw.at[pl.ds(0, WS)])
            pltpu.sync_copy(m_hbm.at[pl.ds(ws, WS)], mw)

        def win_full(w, c2):
            load_win(w)

            def pair(e2, c3):
                update(2 * e2, 0)
                update(2 * e2 + 1, 1)
                return c3

            lax.fori_loop(0, WS // 2, pair, 0)
            return c2

        lax.fori_loop(0, nfull, win_full, 0)
        ne = cnt - nfull * WS

        @pl.when(ne > 0)
        def _():
            load_win(nfull)

            def pair(e2, c3):
                for h in (0, 1):
                    idx = 2 * e2 + h

                    @pl.when(idx < ne)
                    def _():
                        update(idx, h)
                return c3

            lax.fori_loop(0, (ne + 1) // 2, pair, 0)

        return c

    lax.fori_loop(0, SC_NW, chunk, 0)

    def merge(r, c):
        for sl in (pl.ds(0, 16), pl.ds(16, 16)):
            v = jnp.maximum(acc[0, r, sl], acc[1, r, sl])
            acc[0, r, sl] = jnp.where(v > -1.0e37, v, 0.0)
        return c

    lax.fori_loop(0, BINW, merge, 0)
    pltpu.sync_copy(acc.at[0], agg_hbm.at[pl.ds(nlo, BINW)])


def _sc_scatter(m, dst_p, offs, cnts):
    return pl.kernel(
        _scatter_body,
        out_type=jax.ShapeDtypeStruct((NPAD, 32), jnp.float32),
        mesh=_sc_mesh(),
        compiler_params=pltpu.CompilerParams(use_tc_tiling_on_sc=False),
        scratch_types=[
            pltpu.VMEM((SC_NW * NBIN + 16,), jnp.int32),
            pltpu.VMEM((SC_NW * NBIN + 16,), jnp.int32),
            pltpu.VMEM((2, BINW, 32), jnp.float32),
            pltpu.VMEM((WS, 32), jnp.float32),
            pltpu.VMEM((WS + 16,), jnp.int32),
        ],
    )(m, dst_p, offs.reshape(SC_NW * NBIN), cnts.reshape(SC_NW * NBIN))


def _edge_layer(P, Q, src_p, dst_p, offs, cnts, W2, b2):
    g = _sc_gather(P, Q, src_p, dst_p)
    m = _edge_mm(g, W2, b2)
    agg_pad = _sc_scatter(m, dst_p, offs, cnts)
    return agg_pad[:N]


def kernel(x, edge_index, batch, emb_charge, emb_pdgid, emb_frompv, W_cat,
           b_cat, W_cont, b_cont, W_all, b_all, bn_gamma, bn_beta, W1_0, b1_0,
           W2_0, b2_0, W1_1, b1_1, W2_1, b2_1, W_out1, b_out1, W_out2, b_out2):
    W_cat_eff = jnp.concatenate([
        emb_charge @ W_cat[0:8],
        emb_pdgid @ W_cat[8:16],
        emb_frompv @ W_cat[16:24],
    ], axis=0)
    row = lambda b: b.reshape(1, -1)
    src, dst = edge_index[0], edge_index[1]
    src_p, dst_p, offs, cnts = _sc_bin(src, dst)

    h0, s1, s2 = _encode(x, W_cat_eff, row(b_cat), W_cont, row(b_cont),
                         W_all, row(b_all))

    WP0 = W1_0[:32] - W1_0[32:]
    WQ0 = W1_0[32:]
    h, P, Q = _pq_first(h0, s1, s2, row(bn_gamma), row(bn_beta), WP0, WQ0,
                        row(b1_0))
    agg0 = _edge_layer(P, Q, src_p, dst_p, offs, cnts, W2_0, row(b2_0))

    WP1 = W1_1[:32] - W1_1[32:]
    WQ1 = W1_1[32:]
    h, P, Q = _pq_next(h, agg0, WP1, WQ1, row(b1_1))
    agg1 = _edge_layer(P, Q, src_p, dst_p, offs, cnts, W2_1, row(b2_1))

    return _final(h, agg1, W_out1, row(b_out1), W_out2, row(b_out2))


# speedup vs baseline: 2.4979x; 1.0133x over previous
import functools

import jax
import jax.numpy as jnp
from jax import lax
from jax.experimental import pallas as pl
from jax.experimental.pallas import tpu as pltpu
from jax.experimental.pallas import tpu_sc as plsc

N = 50000
E = 800000
H = 32
MESG = 64
PDGS = (1, 2, 11, 13, 22, 130, 211)


def _elu(v):
    return jnp.where(v > 0, v, jnp.exp(jnp.minimum(v, 0.0)) - 1.0)


def _encode_body(x_ref, wcat_ref, bcat_ref, wcont_ref, bcont_ref,
                 wall_ref, ball_ref, h_ref, s1_ref, s2_ref):
    i = pl.program_id(0)
    x = x_ref[...]
    x_cont = x[:, :8]
    pdgv = jnp.abs(x[:, 8:9])
    chv = x[:, 9:10] + 1.0
    fpv = x[:, 10:11]
    oh_ch = jnp.concatenate(
        [(chv == float(v)).astype(jnp.float32) for v in range(3)], axis=1)
    oh_pdg = jnp.concatenate(
        [(pdgv == float(v)).astype(jnp.float32) for v in PDGS], axis=1)
    oh_fp = jnp.concatenate(
        [(fpv == float(v)).astype(jnp.float32) for v in range(4)], axis=1)
    oh = jnp.concatenate([oh_ch, oh_pdg, oh_fp], axis=1)
    emb_cat = _elu(jnp.dot(oh, wcat_ref[...], preferred_element_type=jnp.float32)
                   + bcat_ref[...])
    emb_cont = _elu(jnp.dot(x_cont, wcont_ref[...], preferred_element_type=jnp.float32)
                    + bcont_ref[...])
    hin = jnp.concatenate([emb_cat, emb_cont], axis=1)
    h = _elu(jnp.dot(hin, wall_ref[...], preferred_element_type=jnp.float32)
             + ball_ref[...])
    h_ref[...] = h

    @pl.when(i == 0)
    def _():
        s1_ref[...] = jnp.zeros_like(s1_ref)
        s2_ref[...] = jnp.zeros_like(s2_ref)

    s1_ref[...] += jnp.sum(h, axis=0, keepdims=True)
    s2_ref[...] += jnp.sum(h * h, axis=0, keepdims=True)


def _encode(x, W_cat_eff, b_cat, W_cont, b_cont, W_all, b_all):
    blk = 5000
    grid = N // blk
    full = lambda s: pl.BlockSpec(s, lambda i: (0,) * len(s))
    return pl.pallas_call(
        _encode_body,
        grid=(grid,),
        in_specs=[
            pl.BlockSpec((blk, 11), lambda i: (i, 0)),
            full((14, 16)), full((1, 16)), full((8, 16)), full((1, 16)),
            full((32, 32)), full((1, 32)),
        ],
        out_specs=[
            pl.BlockSpec((blk, 32), lambda i: (i, 0)),
            full((1, 32)), full((1, 32)),
        ],
        out_shape=[
            jax.ShapeDtypeStruct((N, 32), jnp.float32),
            jax.ShapeDtypeStruct((1, 32), jnp.float32),
            jax.ShapeDtypeStruct((1, 32), jnp.float32),
        ],
    )(x, W_cat_eff, b_cat, W_cont, b_cont, W_all, b_all)


def _pq_body(h_ref, s1_ref, s2_ref, g_ref, b_ref, wp_ref, wq_ref, bp_ref,
             h_out_ref, p_ref, q_ref):
    mean = s1_ref[...] / N
    var = s2_ref[...] / N - mean * mean
    inv = g_ref[...] * lax.rsqrt(var + 1e-5)
    h = h_ref[...] * inv + (b_ref[...] - mean * inv)
    h_out_ref[...] = h
    p_ref[...] = jnp.dot(h, wp_ref[...], preferred_element_type=jnp.float32) + bp_ref[...]
    q_ref[...] = jnp.dot(h, wq_ref[...], preferred_element_type=jnp.float32)


def _pq_first(h0, s1, s2, gamma, beta, WP, WQ, b1):
    blk = 5000
    full = lambda s: pl.BlockSpec(s, lambda i: (0,) * len(s))
    return pl.pallas_call(
        _pq_body,
        grid=(N // blk,),
        in_specs=[
            pl.BlockSpec((blk, 32), lambda i: (i, 0)),
            full((1, 32)), full((1, 32)), full((1, 32)), full((1, 32)),
            full((32, 64)), full((32, 64)), full((1, 64)),
        ],
        out_specs=[
            pl.BlockSpec((blk, 32), lambda i: (i, 0)),
            pl.BlockSpec((blk, 64), lambda i: (i, 0)),
            pl.BlockSpec((blk, 64), lambda i: (i, 0)),
        ],
        out_shape=[
            jax.ShapeDtypeStruct((N, 32), jnp.float32),
            jax.ShapeDtypeStruct((N, 64), jnp.float32),
            jax.ShapeDtypeStruct((N, 64), jnp.float32),
        ],
    )(h0, s1, s2, gamma, beta, WP, WQ, b1)


def _pq_next_body(h_ref, agg_ref, wp_ref, wq_ref, bp_ref,
                  h_out_ref, p_ref, q_ref):
    h = h_ref[...] + agg_ref[...]
    h_out_ref[...] = h
    p_ref[...] = jnp.dot(h, wp_ref[...], preferred_element_type=jnp.float32) + bp_ref[...]
    q_ref[...] = jnp.dot(h, wq_ref[...], preferred_element_type=jnp.float32)


def _pq_next(h, agg, WP, WQ, b1):
    blk = 5000
    full = lambda s: pl.BlockSpec(s, lambda i: (0,) * len(s))
    return pl.pallas_call(
        _pq_next_body,
        grid=(N // blk,),
        in_specs=[
            pl.BlockSpec((blk, 32), lambda i: (i, 0)),
            pl.BlockSpec((blk, 32), lambda i: (i, 0)),
            full((32, 64)), full((32, 64)), full((1, 64)),
        ],
        out_specs=[
            pl.BlockSpec((blk, 32), lambda i: (i, 0)),
            pl.BlockSpec((blk, 64), lambda i: (i, 0)),
            pl.BlockSpec((blk, 64), lambda i: (i, 0)),
        ],
        out_shape=[
            jax.ShapeDtypeStruct((N, 32), jnp.float32),
            jax.ShapeDtypeStruct((N, 64), jnp.float32),
            jax.ShapeDtypeStruct((N, 64), jnp.float32),
        ],
    )(h, agg, WP, WQ, b1)


def _mm_body(g_ref, w2_ref, b2_ref, m_ref):
    g = _elu(g_ref[...])
    m_ref[...] = _elu(jnp.dot(g, w2_ref[...],
                              preferred_element_type=jnp.float32) + b2_ref[...])


def _edge_mm(g, W2, b2):
    blk = 6400
    full = lambda s: pl.BlockSpec(s, lambda i: (0,) * len(s))
    return pl.pallas_call(
        _mm_body,
        grid=(EP2 // blk,),
        in_specs=[
            pl.BlockSpec((blk, 64), lambda i: (i, 0)),
            full((64, 32)), full((1, 32)),
        ],
        out_specs=pl.BlockSpec((blk, 32), lambda i: (i, 0)),
        out_shape=jax.ShapeDtypeStruct((EP2, 32), jnp.float32),
    )(g, W2, b2)


def _final_body(h_ref, agg_ref, w1_ref, b1_ref, w2_ref, b2_ref, o_ref):
    h = h_ref[...] + agg_ref[...]
    t = _elu(jnp.dot(h, w1_ref[...], preferred_element_type=jnp.float32) + b1_ref[...])
    o = jnp.dot(t, w2_ref[...], preferred_element_type=jnp.float32) + b2_ref[...]
    o_ref[...] = jax.nn.sigmoid(o)


def _final(h, agg, W_out1, b_out1, W_out2, b_out2):
    blk = 5000
    full = lambda s: pl.BlockSpec(s, lambda i: (0,) * len(s))
    out2 = pl.pallas_call(
        _final_body,
        grid=(N // blk,),
        in_specs=[
            pl.BlockSpec((blk, 32), lambda i: (i, 0)),
            pl.BlockSpec((blk, 32), lambda i: (i, 0)),
            full((32, 16)), full((1, 16)), full((16, 1)), full((1, 1)),
        ],
        out_specs=pl.BlockSpec((blk, 1), lambda i: (i, 0)),
        out_shape=jax.ShapeDtypeStruct((N, 1), jnp.float32),
    )(h, agg, W_out1, b_out1, W_out2, b_out2)
    return out2[:, 0]


SC_NC, SC_NS = 2, 16
SC_NW = SC_NC * SC_NS
C = E // SC_NW
CP = 25600
EPAD = SC_NW * CP
EP2 = EPAD + 6400
EPW = CP
GW = 256
NWIN = EPW // GW
NBIN = 32
BINW = 1563
NPAD = NBIN * BINW
WS = 512
SENT = -3.0e38


def _sc_mesh():
    return plsc.VectorSubcoreMesh(core_axis_name="c", subcore_axis_name="s",
                                  num_cores=SC_NC, num_subcores=SC_NS)


def _wid():
    return lax.axis_index("s") * SC_NC + lax.axis_index("c")


def _iota16():
    return lax.iota(jnp.int32, 16)


def _bin_of(n):
    return lax.shift_right_logical(n * 42936, 26)


def _gather_body(p_hbm, q_hbm, src_hbm, dst_hbm, out_hbm,
                 idx_s, idx_d, rows_p, rows_q, g_buf,
                 sem_i0, sem_i1, sem_p0, sem_p1, sem_q0, sem_q1,
                 sem_o0, sem_o1):
    wid = lax.axis_index("s") * SC_NC + lax.axis_index("c")
    base = wid * EPW
    sem_i = (sem_i0, sem_i1)
    sem_p = (sem_p0, sem_p1)
    sem_q = (sem_q0, sem_q1)
    sem_o = (sem_o0, sem_o1)

    def start_idx(w, s):
        b = base + w * GW
        pltpu.async_copy(src_hbm.at[pl.ds(b, GW)], idx_s.at[s], sem_i[s])
        pltpu.async_copy(dst_hbm.at[pl.ds(b, GW)], idx_d.at[s], sem_i[s])

    def wait_idx(s):
        pltpu.make_async_copy(src_hbm.at[pl.ds(base, GW)], idx_s.at[s],
                              sem_i[s]).wait()
        pltpu.make_async_copy(dst_hbm.at[pl.ds(base, GW)], idx_d.at[s],
                              sem_i[s]).wait()

    def start_rows(s):
        pltpu.async_copy(q_hbm.at[idx_s.at[s]], rows_q.at[s], sem_q[s])
        pltpu.async_copy(p_hbm.at[idx_d.at[s]], rows_p.at[s], sem_p[s])

    def wait_rows(s):
        pltpu.make_async_copy(q_hbm.at[idx_s.at[s]], rows_q.at[s],
                              sem_q[s]).wait()
        pltpu.make_async_copy(p_hbm.at[idx_d.at[s]], rows_p.at[s],
                              sem_p[s]).wait()

    def start_out(w, s):
        b = base + w * GW
        pltpu.async_copy(g_buf.at[s], out_hbm.at[pl.ds(b, GW)], sem_o[s])

    def wait_out(s):
        pltpu.make_async_copy(g_buf.at[s], out_hbm.at[pl.ds(base, GW)],
                              sem_o[s]).wait()

    def compute(s):
        @plsc.parallel_loop(0, GW, unroll=4)
        def _(e):
            for j in range(4):
                sl = pl.ds(j * 16, 16)
                g_buf[s, e, sl] = rows_p[s, e, sl] + rows_q[s, e, sl]

    start_idx(0, 0)
    wait_idx(0)
    start_rows(0)
    start_idx(1, 1)

    def half(w, s, o):
        @pl.when(w < NWIN)
        def _():
            @pl.when(w + 1 < NWIN)
            def _():
                wait_idx(o)
                start_rows(o)

            wait_rows(s)

            @pl.when(w + 2 < NWIN)
            def _():
                start_idx(w + 2, s)

            @pl.when(w >= 2)
            def _():
                wait_out(s)

            compute(s)
            start_out(w, s)

    def body(i, c):
        half(2 * i, 0, 1)
        half(2 * i + 1, 1, 0)
        return c

    lax.fori_loop(0, (NWIN + 1) // 2, body, 0)
    wait_out(0)
    wait_out(1)


def _sc_gather(P, Q, src, dst):
    mesh = plsc.VectorSubcoreMesh(core_axis_name="c", subcore_axis_name="s",
                                  num_cores=SC_NC, num_subcores=SC_NS)
    return pl.kernel(
        _gather_body,
        out_type=jax.ShapeDtypeStruct((EP2, 64), jnp.float32),
        mesh=mesh,
        compiler_params=pltpu.CompilerParams(use_tc_tiling_on_sc=False),
        scratch_types=[
            pltpu.VMEM((2, GW), jnp.int32),
            pltpu.VMEM((2, GW), jnp.int32),
            pltpu.VMEM((2, GW, 64), jnp.float32),
            pltpu.VMEM((2, GW, 64), jnp.float32),
            pltpu.VMEM((2, GW, 64), jnp.float32),
        ] + [pltpu.SemaphoreType.DMA] * 8,
    )(P, Q, src, dst)


_BIN_WINS = ((0, 6400), (6400, 6400), (12800, 6400), (19200, 5800))


def _runs(s):
    i16 = _iota16()
    prev = s[jnp.maximum(i16 - 1, 0)]
    first = (i16 == 0) | (s != prev)
    nxt = s[jnp.minimum(i16 + 1, 15)]
    last = (i16 == 15) | (s != nxt)
    pstart = plsc.cummax(jnp.where(first, i16, 0))
    rank = i16 - pstart
    return first, last, rank


def _bin_body(src_hbm, dst_hbm, srcp_hbm, dstp_hbm, offs_hbm, cnts_hbm,
              dstw, srcw, out_src, out_dst, cnt_ref, cur_ref, obuf, cbuf):
    cw = _wid()
    i16 = _iota16()
    zero16 = i16 * 0
    cbase_raw = cw * C
    cbase_p = cw * CP

    cnt_ref[pl.ds(0, 16)] = zero16
    cnt_ref[pl.ds(16, 16)] = zero16
    cnt_ref[pl.ds(32, 16)] = zero16

    def pre(v, c):
        out_src[pl.ds(v * 16, 16)] = zero16
        out_dst[pl.ds(v * 16, 16)] = zero16
        return c

    lax.fori_loop(0, CP // 16, pre, 0)

    for wb, wlen in _BIN_WINS:
        pltpu.sync_copy(dst_hbm.at[pl.ds(cbase_raw + wb, wlen)],
                        dstw.at[pl.ds(0, wlen)])
        nv = (wlen + 15) // 16

        def cvec(v, c, wb=wb):
            d = dstw[pl.ds(v * 16, 16)]
            b = jnp.where(i16 < (C - wb - v * 16), _bin_of(d), NBIN)
            s, _ = plsc.sort_key_val(b, i16)
            _, last, rank = _runs(s)
            cur = plsc.load_gather(cnt_ref, [s])
            plsc.store_scatter(cnt_ref, [s], cur + rank + 1, mask=last)
            return c

        lax.fori_loop(0, nv, cvec, 0)

    cnt0 = cnt_ref[pl.ds(0, 16)]
    cnt1 = cnt_ref[pl.ds(16, 16)]
    r0 = jnp.bitwise_and(cnt0 + 7, -8)
    r1 = jnp.bitwise_and(cnt1 + 7, -8)
    c0 = plsc.cumsum(r0)
    c1 = plsc.cumsum(r1)
    tot0 = c0[zero16 + 15]
    excl0 = c0 - r0
    excl1 = c1 - r1 + tot0
    end_all = c1[zero16 + 15] + tot0
    cur_ref[pl.ds(0, 16)] = excl0
    cur_ref[pl.ds(16, 16)] = excl1
    cur_ref[pl.ds(32, 16)] = end_all + zero16
    obuf[pl.ds(0, 16)] = excl0 + cbase_p
    obuf[pl.ds(16, 16)] = excl1 + cbase_p
    cbuf[pl.ds(0, 16)] = cnt0
    cbuf[pl.ds(16, 16)] = cnt1
    pltpu.sync_copy(obuf, offs_hbm.at[cw])
    pltpu.sync_copy(cbuf, cnts_hbm.at[cw])

    for wb, wlen in _BIN_WINS:
        pltpu.sync_copy(dst_hbm.at[pl.ds(cbase_raw + wb, wlen)],
                        dstw.at[pl.ds(0, wlen)])
        pltpu.sync_copy(src_hbm.at[pl.ds(cbase_raw + wb, wlen)],
                        srcw.at[pl.ds(0, wlen)])
        nv = (wlen + 15) // 16

        def pvec(v, c, wb=wb):
            d = dstw[pl.ds(v * 16, 16)]
            sv = srcw[pl.ds(v * 16, 16)]
            valid = i16 < (C - wb - v * 16)
            b = jnp.where(valid, _bin_of(d), NBIN)
            s, perm = plsc.sort_key_val(b, i16)
            _, last, rank = _runs(s)
            cur = plsc.load_gather(cur_ref, [s])
            pos = cur + rank
            plsc.store_scatter(out_dst, [pos], jnp.where(valid, d, 0)[perm])
            plsc.store_scatter(out_src, [pos], jnp.where(valid, sv, 0)[perm])
            plsc.store_scatter(cur_ref, [s], pos + 1, mask=last)
            return c

        lax.fori_loop(0, nv, pvec, 0)

    pltpu.sync_copy(out_src, srcp_hbm.at[pl.ds(cbase_p, CP)])
    pltpu.sync_copy(out_dst, dstp_hbm.at[pl.ds(cbase_p, CP)])


def _sc_bin(src, dst):
    return pl.kernel(
        _bin_body,
        out_type=[
            jax.ShapeDtypeStruct((EP2,), jnp.int32),
            jax.ShapeDtypeStruct((EP2,), jnp.int32),
            jax.ShapeDtypeStruct((SC_NW, NBIN), jnp.int32),
            jax.ShapeDtypeStruct((SC_NW, NBIN), jnp.int32),
        ],
        mesh=_sc_mesh(),
        compiler_params=pltpu.CompilerParams(use_tc_tiling_on_sc=False,
                                             needs_layout_passes=False),
        scratch_types=[
            pltpu.VMEM((6400,), jnp.int32),
            pltpu.VMEM((6400,), jnp.int32),
            pltpu.VMEM((CP,), jnp.int32),
            pltpu.VMEM((CP,), jnp.int32),
            pltpu.VMEM((48,), jnp.int32),
            pltpu.VMEM((48,), jnp.int32),
            pltpu.VMEM((32,), jnp.int32),
            pltpu.VMEM((32,), jnp.int32),
        ],
    )(src, dst)


def _scatter_body(m_hbm, dstp_hbm, offs_hbm, cnts_hbm, agg_hbm,
                  offs_v, cnts_v, acc, mw, dstw, sem_w):
    t = _wid()
    nlo = t * BINW
    pltpu.sync_copy(offs_hbm, offs_v.at[pl.ds(0, SC_NW * NBIN)])
    pltpu.sync_copy(cnts_hbm, cnts_v.at[pl.ds(0, SC_NW * NBIN)])
    sent = jnp.float32(SENT) + _iota16() * 0.0

    def init(r, c):
        for h in (0, 1):
            acc[h, r, pl.ds(0, 16)] = sent
            acc[h, r, pl.ds(16, 16)] = sent
        return c

    lax.fori_loop(0, BINW, init, 0)

    def update(idx, h):
        rel = dstw[pl.ds(idx, 16)][0] - nlo
        a0 = acc[h, rel, pl.ds(0, 16)]
        a1 = acc[h, rel, pl.ds(16, 16)]
        acc[h, rel, pl.ds(0, 16)] = jnp.maximum(a0, mw[idx, pl.ds(0, 16)])
        acc[h, rel, pl.ds(16, 16)] = jnp.maximum(a1, mw[idx, pl.ds(16, 16)])

    def chunk(ci, c):
        off = offs_v[pl.ds(ci * NBIN + t, 16)][0]
        cnt = cnts_v[pl.ds(ci * NBIN + t, 16)][0]
        nfull = cnt // WS

        def load_win(w):
            ws = pl.multiple_of(off + w * WS, 8)
            c1 = pltpu.async_copy(dstp_hbm.at[pl.ds(ws, WS)],
                                  dstw.at[pl.ds(0, WS)], sem_w)
            c2 = pltpu.async_copy(m_hbm.at[pl.ds(ws, WS)], mw, sem_w)
            c1.wait()
            c2.wait()

        def win_full(w, c2):
            load_win(w)

            def pair(e2, c3):
                update(2 * e2, 0)
                update(2 * e2 + 1, 1)
                return c3

            lax.fori_loop(0, WS // 2, pair, 0)
            return c2

        lax.fori_loop(0, nfull, win_full, 0)
        ne = cnt - nfull * WS

        @pl.when(ne > 0)
        def _():
            load_win(nfull)

            def pair(e2, c3):
                for h in (0, 1):
                    idx = 2 * e2 + h

                    @pl.when(idx < ne)
                    def _():
                        update(idx, h)
                return c3

            lax.fori_loop(0, (ne + 1) // 2, pair, 0)

        return c

    lax.fori_loop(0, SC_NW, chunk, 0)

    def merge(r, c):
        for sl in (pl.ds(0, 16), pl.ds(16, 16)):
            v = jnp.maximum(acc[0, r, sl], acc[1, r, sl])
            acc[0, r, sl] = jnp.where(v > -1.0e37, v, 0.0)
        return c

    lax.fori_loop(0, BINW, merge, 0)
    pltpu.sync_copy(acc.at[0], agg_hbm.at[pl.ds(nlo, BINW)])


def _sc_scatter(m, dst_p, offs, cnts):
    return pl.kernel(
        _scatter_body,
        out_type=jax.ShapeDtypeStruct((NPAD, 32), jnp.float32),
        mesh=_sc_mesh(),
        compiler_params=pltpu.CompilerParams(use_tc_tiling_on_sc=False),
        scratch_types=[
            pltpu.VMEM((SC_NW * NBIN + 16,), jnp.int32),
            pltpu.VMEM((SC_NW * NBIN + 16,), jnp.int32),
            pltpu.VMEM((2, BINW, 32), jnp.float32),
            pltpu.VMEM((WS, 32), jnp.float32),
            pltpu.VMEM((WS + 16,), jnp.int32),
            pltpu.SemaphoreType.DMA,
        ],
    )(m, dst_p, offs.reshape(SC_NW * NBIN), cnts.reshape(SC_NW * NBIN))


def _edge_layer(P, Q, src_p, dst_p, offs, cnts, W2, b2):
    g = _sc_gather(P, Q, src_p, dst_p)
    m = _edge_mm(g, W2, b2)
    agg_pad = _sc_scatter(m, dst_p, offs, cnts)
    return agg_pad[:N]


def kernel(x, edge_index, batch, emb_charge, emb_pdgid, emb_frompv, W_cat,
           b_cat, W_cont, b_cont, W_all, b_all, bn_gamma, bn_beta, W1_0, b1_0,
           W2_0, b2_0, W1_1, b1_1, W2_1, b2_1, W_out1, b_out1, W_out2, b_out2):
    W_cat_eff = jnp.concatenate([
        emb_charge @ W_cat[0:8],
        emb_pdgid @ W_cat[8:16],
        emb_frompv @ W_cat[16:24],
    ], axis=0)
    row = lambda b: b.reshape(1, -1)
    src, dst = edge_index[0], edge_index[1]
    src_p, dst_p, offs, cnts = _sc_bin(src, dst)

    h0, s1, s2 = _encode(x, W_cat_eff, row(b_cat), W_cont, row(b_cont),
                         W_all, row(b_all))

    WP0 = W1_0[:32] - W1_0[32:]
    WQ0 = W1_0[32:]
    h, P, Q = _pq_first(h0, s1, s2, row(bn_gamma), row(bn_beta), WP0, WQ0,
                        row(b1_0))
    agg0 = _edge_layer(P, Q, src_p, dst_p, offs, cnts, W2_0, row(b2_0))

    WP1 = W1_1[:32] - W1_1[32:]
    WQ1 = W1_1[32:]
    h, P, Q = _pq_next(h, agg0, WP1, WQ1, row(b1_1))
    agg1 = _edge_layer(P, Q, src_p, dst_p, offs, cnts, W2_1, row(b2_1))

    return _final(h, agg1, W_out1, row(b_out1), W_out2, row(b_out2))
